# baseline (speedup 1.0000x reference)
import functools
import jax
import jax.numpy as jnp
from jax import lax
from jax.experimental import pallas as pl
from jax.experimental.pallas import tpu as pltpu
from jax.experimental.pallas import tpu_sc as plsc

N = 10000
E = 320000
HC = 128
ENF = 16
NG = 16
PD = 16
HD = HC + PD
NC = 2
NS = 16
NW = NC * NS
EPW = E // NW
CH = 80
NPT = N // NS

BN = 2000
BE = 2000
HB = 160

f32 = jnp.float32
bf16 = jnp.bfloat16


def _swish(x):
    return x * jax.nn.sigmoid(x)


def _dot(a, b):
    return jnp.dot(a, b, preferred_element_type=f32)


def _pre_body(feat_ref, pos_ref, t_ref, w1, b1, w2, b2, ew, e127, eb,
              out_ref, out16_ref):
    z = _swish(_dot(feat_ref[...], w1[...]) + b1[...])
    hp = _dot(z, w2[...]) + b2[...]
    h0 = _dot(hp, ew[...]) + t_ref[0, 0] * e127[...] + eb[...]
    out_ref[:, :HC] = h0
    out_ref[:, HC:] = pos_ref[...]
    out16_ref[:, :HC] = h0.astype(bf16)
    out16_ref[:, HC:HD] = pos_ref[...].astype(bf16)
    out16_ref[:, HD:] = jnp.zeros((out16_ref.shape[0], HB - HD), bf16)


def _pre_call(feat, pos_pad, t2, w1, b1, w2p, b2p, ew, e127, eb):
    g = N // BN
    const = lambda shape: pl.BlockSpec(shape, lambda i: (0, 0))
    return pl.pallas_call(
        _pre_body,
        grid=(g,),
        in_specs=[
            pl.BlockSpec((BN, HC), lambda i: (i, 0)),
            pl.BlockSpec((BN, PD), lambda i: (i, 0)),
            pl.BlockSpec(memory_space=pltpu.SMEM),
            const((HC, 256)), const((1, 256)),
            const((256, HC)), const((1, HC)),
            const((HC, HC)), const((1, HC)), const((1, HC)),
        ],
        out_specs=[pl.BlockSpec((BN, HD), lambda i: (i, 0)),
                   pl.BlockSpec((BN, HB), lambda i: (i, 0))],
        out_shape=[jax.ShapeDtypeStruct((N, HD), f32),
                   jax.ShapeDtypeStruct((N, HB), bf16)],
    )(feat, pos_pad, t2, w1, b1, w2p, b2p, ew, e127, eb)


CHG = 200
NITG = EPW // CHG


def _gather_call(hcat, src_i, dst_i):
    mesh = plsc.VectorSubcoreMesh(core_axis_name="c", subcore_axis_name="s")

    @functools.partial(
        pl.kernel,
        out_type=(jax.ShapeDtypeStruct((E, HB), bf16),
                  jax.ShapeDtypeStruct((E, HB), bf16)),
        mesh=mesh,
        scratch_types=(pltpu.VMEM((EPW,), jnp.int32),
                       pltpu.VMEM((2, CHG, HB), bf16),
                       pltpu.SemaphoreType.DMA((2,)),
                       pltpu.SemaphoreType.DMA((2,))),
        compiler_params=pltpu.CompilerParams(use_tc_tiling_on_sc=False),
    )
    def gather_k(hcat_ref, src_ref, dst_ref, osrc_ref, odst_ref,
                 idxb, buf, gsem, osem):
        wid = lax.axis_index("s") * NC + lax.axis_index("c")
        base0 = wid * EPW

        def phase(idx_hbm, out_hbm):
            pltpu.sync_copy(idx_hbm.at[pl.ds(base0, EPW)], idxb)

            def g_start(k, p):
                pltpu.async_copy(
                    hcat_ref.at[idxb.at[pl.ds(k * CHG, CHG)]],
                    buf.at[p], gsem.at[p])

            def g_wait(k, p):
                pltpu.make_async_copy(
                    hcat_ref.at[idxb.at[pl.ds(k * CHG, CHG)]],
                    buf.at[p], gsem.at[p]).wait()

            def o_start(k, p):
                pltpu.async_copy(
                    buf.at[p], out_hbm.at[pl.ds(base0 + k * CHG, CHG)],
                    osem.at[p])

            def o_wait(k):
                p = k % 2
                pltpu.make_async_copy(
                    buf.at[p], out_hbm.at[pl.ds(base0 + k * CHG, CHG)],
                    osem.at[p]).wait()

            g_start(0, 0)

            def body(k, carry):
                p = k % 2

                @pl.when(k + 1 < NITG)
                def _():
                    @pl.when(k >= 1)
                    def _():
                        o_wait(k - 1)
                    g_start(k + 1, 1 - p)

                g_wait(k, p)
                o_start(k, p)
                return carry

            lax.fori_loop(0, NITG, body, 0)
            o_wait(NITG - 2)
            o_wait(NITG - 1)

        phase(src_ref, osrc_ref)
        phase(dst_ref, odst_ref)

    return gather_k(hcat, src_i, dst_i)


def _edge_body(gs_ref, gd_ref, ea_ref, eew1, eeb1, eew2, eeb2,
               w1s, w1d, w1e, w1d2, b1, w2, b2, xw, xb, out_ref):
    hs = gs_ref[:, :HC].astype(f32)
    hd = gd_ref[:, :HC].astype(f32)
    rel = (gs_ref[:, HC:HD].astype(f32)
           - gd_ref[:, HC:HD].astype(f32))
    d2 = jnp.sum(rel * rel, axis=1, keepdims=True)
    e = _dot(_swish(_dot(ea_ref[...], eew1[...]) + eeb1[...]), eew2[...]) + eeb2[...]
    pre = (_dot(hs, w1s[...]) + _dot(hd, w1d[...]) + _dot(e, w1e[...])
           + d2 * w1d2[...] + b1[...])
    m = _swish(_dot(_swish(pre), w2[...]) + b2[...])
    coef = _dot(m, xw[...]) + xb[...]
    out_ref[:, :HC] = m
    out_ref[:, HC:] = rel * (coef / (jnp.sqrt(d2) + 1.0))


def _edge_call(gsrc, gdst, edge_attr, eew1, eeb1, eew2, eeb2,
               w1s, w1d, w1e, w1d2, b1, w2, b2, xw, xb):
    g = E // BE
    const = lambda shape: pl.BlockSpec(shape, lambda i: (0, 0))
    return pl.pallas_call(
        _edge_body,
        grid=(g,),
        in_specs=[
            pl.BlockSpec((BE, HB), lambda i: (i, 0)),
            pl.BlockSpec((BE, HB), lambda i: (i, 0)),
            pl.BlockSpec((BE, ENF), lambda i: (i, 0)),
            const((ENF, 2 * ENF)), const((1, 2 * ENF)),
            const((2 * ENF, ENF)), const((1, ENF)),
            const((HC, HC)), const((HC, HC)), const((ENF, HC)),
            const((1, HC)), const((1, HC)),
            const((HC, HC)), const((1, HC)),
            const((HC, 1)), const((1, 1)),
        ],
        out_specs=pl.BlockSpec((BE, HD), lambda i: (i, 0)),
        out_shape=jax.ShapeDtypeStruct((E, HD), f32),
    )(gsrc, gdst, edge_attr, eew1, eeb1, eew2, eeb2,
      w1s, w1d, w1e, w1d2, b1, w2, b2, xw, xb)


def _scatter_call(mw, dst_i, zrows):
    mesh = plsc.VectorSubcoreMesh(core_axis_name="c", subcore_axis_name="s")

    @functools.partial(
        pl.kernel,
        out_type=jax.ShapeDtypeStruct((NC * N, HD), f32),
        mesh=mesh,
        scratch_types=(pltpu.VMEM((2, CH), jnp.int32),
                       pltpu.VMEM((2, CH, HD), f32),
                       pltpu.SemaphoreType.DMA((2,)),
                       pltpu.SemaphoreType.DMA((2,)),
                       pltpu.VMEM_SHARED((N, HD), f32)),
        compiler_params=pltpu.CompilerParams(use_tc_tiling_on_sc=False),
    )
    def scatter_k(mw_ref, dst_ref, z_ref, out_ref, ib, rb, lsem_i, lsem_r, acc):
        c = lax.axis_index("c")
        s = lax.axis_index("s")
        wid = s * NC + c
        pltpu.sync_copy(z_ref, acc.at[pl.ds(s * NPT, NPT)])
        plsc.subcore_barrier()
        base0 = wid * EPW
        nit = EPW // CH

        def l_start(k, p):
            b = base0 + k * CH
            pltpu.async_copy(dst_ref.at[pl.ds(b, CH)], ib.at[p], lsem_i.at[p])
            pltpu.async_copy(mw_ref.at[pl.ds(b, CH)], rb.at[p], lsem_r.at[p])

        def l_wait(k, p):
            b = base0 + k * CH
            pltpu.make_async_copy(dst_ref.at[pl.ds(b, CH)], ib.at[p],
                                  lsem_i.at[p]).wait()
            pltpu.make_async_copy(mw_ref.at[pl.ds(b, CH)], rb.at[p],
                                  lsem_r.at[p]).wait()

        l_start(0, 0)

        def body(k, carry):
            p = k % 2

            @pl.when(k + 1 < nit)
            def _():
                l_start(k + 1, 1 - p)

            l_wait(k, p)
            pltpu.sync_copy(rb.at[p], acc.at[ib.at[p]], add=True)
            return carry

        lax.fori_loop(0, nit, body, 0)
        plsc.subcore_barrier()
        pltpu.sync_copy(acc.at[pl.ds(s * NPT, NPT)],
                        out_ref.at[pl.ds(c * N + s * NPT, NPT)])

    return scatter_k(mw, dst_i, zrows)


def _node_body(hc_ref, agg_ref, w1h, w1a, b1, w2, b2, out_ref, out16_ref):
    hc = hc_ref[...]
    h = hc[:, :HC]
    ag = agg_ref[0] + agg_ref[1]
    u = _swish(_dot(h, w1h[...]) + _dot(ag[:, :HC], w1a[...]) + b1[...])
    hn = h + _dot(u, w2[...]) + b2[...]
    pn = hc[:, HC:] + ag[:, HC:]
    out_ref[:, :HC] = hn
    out_ref[:, HC:] = pn
    out16_ref[:, :HC] = hn.astype(bf16)
    out16_ref[:, HC:HD] = pn.astype(bf16)
    out16_ref[:, HD:] = jnp.zeros((out16_ref.shape[0], HB - HD), bf16)


def _node_call(hcat, aggp, w1h, w1a, b1, w2, b2):
    g = N // BN
    const = lambda shape: pl.BlockSpec(shape, lambda i: (0, 0))
    return pl.pallas_call(
        _node_body,
        grid=(g,),
        in_specs=[
            pl.BlockSpec((BN, HD), lambda i: (i, 0)),
            pl.BlockSpec((NC, BN, HD), lambda i: (0, i, 0)),
            const((HC, HC)), const((HC, HC)), const((1, HC)),
            const((HC, HC)), const((1, HC)),
        ],
        out_specs=[pl.BlockSpec((BN, HD), lambda i: (i, 0)),
                   pl.BlockSpec((BN, HB), lambda i: (i, 0))],
        out_shape=[jax.ShapeDtypeStruct((N, HD), f32),
                   jax.ShapeDtypeStruct((N, HB), bf16)],
    )(hcat, aggp, w1h, w1a, b1, w2, b2)


def _ro_body(hc_ref, mk_ref, w1, b1, w1g, b1g, w2, b2, w2g, b2g, w3, b3,
             conf_ref, s_acc, c_acc):
    i = pl.program_id(0)

    @pl.when(i == 0)
    def _():
        s_acc[...] = jnp.zeros_like(s_acc)
        c_acc[...] = jnp.zeros_like(c_acc)

    h = hc_ref[:, :HC]
    g1 = jax.nn.sigmoid(_dot(h, w1g[...]) + b1g[...])
    v = _swish((_dot(h, w1[...]) + b1[...]) * g1)
    g2 = jax.nn.sigmoid(_dot(v, w2g[...]) + b2g[...])
    v = _swish((_dot(v, w2[...]) + b2[...]) * g2)
    nout = _dot(v, w3[...]) + b3[...]
    oh = (mk_ref[...] == lax.broadcasted_iota(jnp.int32, (BN, NG), 1)).astype(f32)
    s_acc[...] += jnp.sum(oh * nout, axis=0, keepdims=True)
    c_acc[...] += jnp.sum(oh, axis=0, keepdims=True)
    conf_ref[...] = s_acc[...] / jnp.maximum(c_acc[...], 1.0)


def _ro_call(hcat, mask2, w1, b1, w1g, b1g, w2, b2, w2g, b2g, w3, b3):
    g = N // BN
    const = lambda shape: pl.BlockSpec(shape, lambda i: (0, 0))
    return pl.pallas_call(
        _ro_body,
        grid=(g,),
        in_specs=[
            pl.BlockSpec((BN, HD), lambda i: (i, 0)),
            pl.BlockSpec((BN, 1), lambda i: (i, 0)),
            const((HC, HC)), const((1, HC)),
            const((HC, HC)), const((1, HC)),
            const((HC, HC)), const((1, HC)),
            const((HC, HC)), const((1, HC)),
            const((HC, 1)), const((1, 1)),
        ],
        out_specs=pl.BlockSpec((1, NG), lambda i: (0, 0)),
        out_shape=jax.ShapeDtypeStruct((1, NG), f32),
        scratch_shapes=[pltpu.VMEM((1, NG), f32), pltpu.VMEM((1, NG), f32)],
    )(hcat, mask2, w1, b1, w1g, b1g, w2, b2, w2g, b2g, w3, b3)


def kernel(xh0, edge_index, t, conditions, n_frag_switch, combined_mask,
           edge_attr, params):
    p = params
    feat = xh0[:, 3:]
    pos_pad = jnp.pad(xh0[:, :3], ((0, 0), (0, PD - 3)))
    t2 = t.reshape(1, 1)
    src = edge_index[0]
    dst = edge_index[1]

    w2p = jnp.pad(p['enc_W2'], ((0, 0), (0, 1)))
    b2p = jnp.pad(p['enc_b2'], (0, 1)).reshape(1, HC)
    e127 = p['emb_W'][HC - 1:HC, :]

    hcat, hcat16 = _pre_call(feat, pos_pad, t2,
                             p['enc_W1'], p['enc_b1'].reshape(1, 256),
                             w2p, b2p,
                             p['emb_W'], e127, p['emb_b'].reshape(1, HC))

    zrows = jnp.zeros((NPT, HD), f32)
    for l in range(2):
        ew1 = p['l%d_eW1' % l]
        gsrc, gdst = _gather_call(hcat16, src, dst)
        mw = _edge_call(gsrc, gdst, edge_attr,
                        p['ee_W1'], p['ee_b1'].reshape(1, 2 * ENF),
                        p['ee_W2'], p['ee_b2'].reshape(1, ENF),
                        ew1[:HC], ew1[HC:2 * HC], ew1[2 * HC + 1:],
                        ew1[2 * HC:2 * HC + 1],
                        p['l%d_eb1' % l].reshape(1, HC),
                        p['l%d_eW2' % l], p['l%d_eb2' % l].reshape(1, HC),
                        p['l%d_xW' % l], p['l%d_xb' % l].reshape(1, 1))
        aggf = _scatter_call(mw, dst, zrows)
        hw1 = p['l%d_hW1' % l]
        hcat, hcat16 = _node_call(hcat, aggf.reshape(NC, N, HD),
                          hw1[:HC], hw1[HC:],
                          p['l%d_hb1' % l].reshape(1, HC),
                          p['l%d_hW2' % l], p['l%d_hb2' % l].reshape(1, HC))

    conf = _ro_call(hcat, combined_mask.reshape(N, 1),
                    p['ro_W1'], p['ro_b1'].reshape(1, HC),
                    p['ro_W1g'], p['ro_b1g'].reshape(1, HC),
                    p['ro_W2'], p['ro_b2'].reshape(1, HC),
                    p['ro_W2g'], p['ro_b2g'].reshape(1, HC),
                    p['ro_W3'], p['ro_b3'].reshape(1, 1))
    return conf.reshape(NG, 1)


# speedup vs baseline: 1.0426x; 1.0426x over previous
import functools
import jax
import jax.numpy as jnp
from jax import lax
from jax.experimental import pallas as pl
from jax.experimental.pallas import tpu as pltpu
from jax.experimental.pallas import tpu_sc as plsc

N = 10000
E = 320000
HC = 128
ENF = 16
NG = 16
PD = 16
HD = HC + PD
NC = 2
NS = 16
NW = NC * NS
EPW = E // NW
CH = 80
NPT = N // NS

BN = 2000
BE = 2000
PB = 32

f32 = jnp.float32
bf16 = jnp.bfloat16


def _swish(x):
    return x * jax.nn.sigmoid(x)


def _dot(a, b):
    return jnp.dot(a, b, preferred_element_type=f32)


def _pre_body(feat_ref, pos_ref, t_ref, w1, b1, w2, b2, ew, e127, eb,
              out_ref, h16_ref, p16_ref):
    z = _swish(_dot(feat_ref[...], w1[...]) + b1[...])
    hp = _dot(z, w2[...]) + b2[...]
    h0 = _dot(hp, ew[...]) + t_ref[0, 0] * e127[...] + eb[...]
    out_ref[:, :HC] = h0
    out_ref[:, HC:] = pos_ref[...]
    h16_ref[...] = h0.astype(bf16)
    p16_ref[:, :PD] = pos_ref[...].astype(bf16)
    p16_ref[:, PD:] = jnp.zeros((p16_ref.shape[0], PB - PD), bf16)


def _pre_call(feat, pos_pad, t2, w1, b1, w2p, b2p, ew, e127, eb):
    g = N // BN
    const = lambda shape: pl.BlockSpec(shape, lambda i: (0, 0))
    return pl.pallas_call(
        _pre_body,
        grid=(g,),
        in_specs=[
            pl.BlockSpec((BN, HC), lambda i: (i, 0)),
            pl.BlockSpec((BN, PD), lambda i: (i, 0)),
            pl.BlockSpec(memory_space=pltpu.SMEM),
            const((HC, 256)), const((1, 256)),
            const((256, HC)), const((1, HC)),
            const((HC, HC)), const((1, HC)), const((1, HC)),
        ],
        out_specs=[pl.BlockSpec((BN, HD), lambda i: (i, 0)),
                   pl.BlockSpec((BN, HC), lambda i: (i, 0)),
                   pl.BlockSpec((BN, PB), lambda i: (i, 0))],
        out_shape=[jax.ShapeDtypeStruct((N, HD), f32),
                   jax.ShapeDtypeStruct((N, HC), bf16),
                   jax.ShapeDtypeStruct((N, PB), bf16)],
    )(feat, pos_pad, t2, w1, b1, w2p, b2p, ew, e127, eb)


CHG = 200
NITG = EPW // CHG


def _gather_call(h16, p16, src_i, dst_i):
    mesh = plsc.VectorSubcoreMesh(core_axis_name="c", subcore_axis_name="s")

    @functools.partial(
        pl.kernel,
        out_type=(jax.ShapeDtypeStruct((E, HC), bf16),
                  jax.ShapeDtypeStruct((E, PB), bf16),
                  jax.ShapeDtypeStruct((E, HC), bf16),
                  jax.ShapeDtypeStruct((E, PB), bf16)),
        mesh=mesh,
        scratch_types=(pltpu.VMEM((EPW,), jnp.int32),
                       pltpu.VMEM((2, CHG, HC), bf16),
                       pltpu.VMEM((2, CHG, PB), bf16),
                       pltpu.SemaphoreType.DMA((2,)),
                       pltpu.SemaphoreType.DMA((2,))),
        compiler_params=pltpu.CompilerParams(use_tc_tiling_on_sc=False),
    )
    def gather_k(h16_ref, p16_ref, src_ref, dst_ref,
                 ohs_ref, ops_ref, ohd_ref, opd_ref,
                 idxb, bh, bp, gsem, osem):
        wid = lax.axis_index("s") * NC + lax.axis_index("c")
        base0 = wid * EPW

        def phase(idx_hbm, oh_hbm, op_hbm):
            pltpu.sync_copy(idx_hbm.at[pl.ds(base0, EPW)], idxb)

            def descs(k, p):
                isl = idxb.at[pl.ds(k * CHG, CHG)]
                osl = pl.ds(base0 + k * CHG, CHG)
                return (
                    pltpu.make_async_copy(h16_ref.at[isl], bh.at[p], gsem.at[p]),
                    pltpu.make_async_copy(p16_ref.at[isl], bp.at[p], gsem.at[p]),
                    pltpu.make_async_copy(bh.at[p], oh_hbm.at[osl], osem.at[p]),
                    pltpu.make_async_copy(bp.at[p], op_hbm.at[osl], osem.at[p]),
                )

            def g_start(k, p):
                gh, gp, _, _ = descs(k, p)
                gh.start()
                gp.start()

            def g_wait(k, p):
                gh, gp, _, _ = descs(k, p)
                gh.wait()
                gp.wait()

            def o_start(k, p):
                _, _, oh, op = descs(k, p)
                oh.start()
                op.start()

            def o_wait(k):
                _, _, oh, op = descs(k, k % 2)
                oh.wait()
                op.wait()

            g_start(0, 0)

            def body(k, carry):
                p = k % 2

                @pl.when(k + 1 < NITG)
                def _():
                    @pl.when(k >= 1)
                    def _():
                        o_wait(k - 1)
                    g_start(k + 1, 1 - p)

                g_wait(k, p)
                o_start(k, p)
                return carry

            lax.fori_loop(0, NITG, body, 0)
            o_wait(NITG - 2)
            o_wait(NITG - 1)

        phase(src_ref, ohs_ref, ops_ref)
        phase(dst_ref, ohd_ref, opd_ref)

    return gather_k(h16, p16, src_i, dst_i)


def _edge_body(hs_ref, ps_ref, hd_ref, pd_ref, ea_ref, eew1, eeb1, eew2, eeb2,
               w1s, w1d, w1e, w1d2, b1, w2, b2, xw, xb, out_ref):
    rel = (ps_ref[...].astype(f32)
           - pd_ref[...].astype(f32))
    d2 = jnp.sum(rel * rel, axis=1, keepdims=True)
    e = _dot(_swish(_dot(ea_ref[...], eew1[...]) + eeb1[...]), eew2[...]) + eeb2[...]
    pre = (_dot(hs_ref[...], w1s[...]) + _dot(hd_ref[...], w1d[...])
           + _dot(e, w1e[...]) + d2 * w1d2[...] + b1[...])
    m = _swish(_dot(_swish(pre), w2[...]) + b2[...])
    coef = _dot(m, xw[...]) + xb[...]
    out_ref[:, :HC] = m
    out_ref[:, HC:] = rel[:, :PD] * (coef / (jnp.sqrt(d2) + 1.0))


def _edge_call(hs16, ps16, hd16, pd16, edge_attr, eew1, eeb1, eew2, eeb2,
               w1s, w1d, w1e, w1d2, b1, w2, b2, xw, xb):
    g = E // BE
    const = lambda shape: pl.BlockSpec(shape, lambda i: (0, 0))
    return pl.pallas_call(
        _edge_body,
        grid=(g,),
        in_specs=[
            pl.BlockSpec((BE, HC), lambda i: (i, 0)),
            pl.BlockSpec((BE, PB), lambda i: (i, 0)),
            pl.BlockSpec((BE, HC), lambda i: (i, 0)),
            pl.BlockSpec((BE, PB), lambda i: (i, 0)),
            pl.BlockSpec((BE, ENF), lambda i: (i, 0)),
            const((ENF, 2 * ENF)), const((1, 2 * ENF)),
            const((2 * ENF, ENF)), const((1, ENF)),
            const((HC, HC)), const((HC, HC)), const((ENF, HC)),
            const((1, HC)), const((1, HC)),
            const((HC, HC)), const((1, HC)),
            const((HC, 1)), const((1, 1)),
        ],
        out_specs=pl.BlockSpec((BE, HD), lambda i: (i, 0)),
        out_shape=jax.ShapeDtypeStruct((E, HD), f32),
    )(hs16, ps16, hd16, pd16, edge_attr, eew1, eeb1, eew2, eeb2,
      w1s, w1d, w1e, w1d2, b1, w2, b2, xw, xb)


def _scatter_call(mw, dst_i, zrows):
    mesh = plsc.VectorSubcoreMesh(core_axis_name="c", subcore_axis_name="s")

    @functools.partial(
        pl.kernel,
        out_type=jax.ShapeDtypeStruct((NC * N, HD), f32),
        mesh=mesh,
        scratch_types=(pltpu.VMEM((2, CH), jnp.int32),
                       pltpu.VMEM((2, CH, HD), f32),
                       pltpu.SemaphoreType.DMA((2,)),
                       pltpu.SemaphoreType.DMA((2,)),
                       pltpu.VMEM_SHARED((N, HD), f32)),
        compiler_params=pltpu.CompilerParams(use_tc_tiling_on_sc=False),
    )
    def scatter_k(mw_ref, dst_ref, z_ref, out_ref, ib, rb, lsem_i, lsem_r, acc):
        c = lax.axis_index("c")
        s = lax.axis_index("s")
        wid = s * NC + c
        pltpu.sync_copy(z_ref, acc.at[pl.ds(s * NPT, NPT)])
        plsc.subcore_barrier()
        base0 = wid * EPW
        nit = EPW // CH

        def l_start(k, p):
            b = base0 + k * CH
            pltpu.async_copy(dst_ref.at[pl.ds(b, CH)], ib.at[p], lsem_i.at[p])
            pltpu.async_copy(mw_ref.at[pl.ds(b, CH)], rb.at[p], lsem_r.at[p])

        def l_wait(k, p):
            b = base0 + k * CH
            pltpu.make_async_copy(dst_ref.at[pl.ds(b, CH)], ib.at[p],
                                  lsem_i.at[p]).wait()
            pltpu.make_async_copy(mw_ref.at[pl.ds(b, CH)], rb.at[p],
                                  lsem_r.at[p]).wait()

        l_start(0, 0)

        def body(k, carry):
            p = k % 2

            @pl.when(k + 1 < nit)
            def _():
                l_start(k + 1, 1 - p)

            l_wait(k, p)
            pltpu.sync_copy(rb.at[p], acc.at[ib.at[p]], add=True)
            return carry

        lax.fori_loop(0, nit, body, 0)
        plsc.subcore_barrier()
        pltpu.sync_copy(acc.at[pl.ds(s * NPT, NPT)],
                        out_ref.at[pl.ds(c * N + s * NPT, NPT)])

    return scatter_k(mw, dst_i, zrows)


def _node_body(hc_ref, agg_ref, w1h, w1a, b1, w2, b2,
               out_ref, h16_ref, p16_ref):
    hc = hc_ref[...]
    h = hc[:, :HC]
    ag = agg_ref[0] + agg_ref[1]
    u = _swish(_dot(h, w1h[...]) + _dot(ag[:, :HC], w1a[...]) + b1[...])
    hn = h + _dot(u, w2[...]) + b2[...]
    pn = hc[:, HC:] + ag[:, HC:]
    out_ref[:, :HC] = hn
    out_ref[:, HC:] = pn
    h16_ref[...] = hn.astype(bf16)
    p16_ref[:, :PD] = pn.astype(bf16)
    p16_ref[:, PD:] = jnp.zeros((p16_ref.shape[0], PB - PD), bf16)


def _node_call(hcat, aggp, w1h, w1a, b1, w2, b2):
    g = N // BN
    const = lambda shape: pl.BlockSpec(shape, lambda i: (0, 0))
    return pl.pallas_call(
        _node_body,
        grid=(g,),
        in_specs=[
            pl.BlockSpec((BN, HD), lambda i: (i, 0)),
            pl.BlockSpec((NC, BN, HD), lambda i: (0, i, 0)),
            const((HC, HC)), const((HC, HC)), const((1, HC)),
            const((HC, HC)), const((1, HC)),
        ],
        out_specs=[pl.BlockSpec((BN, HD), lambda i: (i, 0)),
                   pl.BlockSpec((BN, HC), lambda i: (i, 0)),
                   pl.BlockSpec((BN, PB), lambda i: (i, 0))],
        out_shape=[jax.ShapeDtypeStruct((N, HD), f32),
                   jax.ShapeDtypeStruct((N, HC), bf16),
                   jax.ShapeDtypeStruct((N, PB), bf16)],
    )(hcat, aggp, w1h, w1a, b1, w2, b2)


def _ro_body(hc_ref, mk_ref, w1, b1, w1g, b1g, w2, b2, w2g, b2g, w3, b3,
             conf_ref, s_acc, c_acc):
    i = pl.program_id(0)

    @pl.when(i == 0)
    def _():
        s_acc[...] = jnp.zeros_like(s_acc)
        c_acc[...] = jnp.zeros_like(c_acc)

    h = hc_ref[:, :HC]
    g1 = jax.nn.sigmoid(_dot(h, w1g[...]) + b1g[...])
    v = _swish((_dot(h, w1[...]) + b1[...]) * g1)
    g2 = jax.nn.sigmoid(_dot(v, w2g[...]) + b2g[...])
    v = _swish((_dot(v, w2[...]) + b2[...]) * g2)
    nout = _dot(v, w3[...]) + b3[...]
    oh = (mk_ref[...] == lax.broadcasted_iota(jnp.int32, (BN, NG), 1)).astype(f32)
    s_acc[...] += jnp.sum(oh * nout, axis=0, keepdims=True)
    c_acc[...] += jnp.sum(oh, axis=0, keepdims=True)
    conf_ref[...] = s_acc[...] / jnp.maximum(c_acc[...], 1.0)


def _ro_call(hcat, mask2, w1, b1, w1g, b1g, w2, b2, w2g, b2g, w3, b3):
    g = N // BN
    const = lambda shape: pl.BlockSpec(shape, lambda i: (0, 0))
    return pl.pallas_call(
        _ro_body,
        grid=(g,),
        in_specs=[
            pl.BlockSpec((BN, HD), lambda i: (i, 0)),
            pl.BlockSpec((BN, 1), lambda i: (i, 0)),
            const((HC, HC)), const((1, HC)),
            const((HC, HC)), const((1, HC)),
            const((HC, HC)), const((1, HC)),
            const((HC, HC)), const((1, HC)),
            const((HC, 1)), const((1, 1)),
        ],
        out_specs=pl.BlockSpec((1, NG), lambda i: (0, 0)),
        out_shape=jax.ShapeDtypeStruct((1, NG), f32),
        scratch_shapes=[pltpu.VMEM((1, NG), f32), pltpu.VMEM((1, NG), f32)],
    )(hcat, mask2, w1, b1, w1g, b1g, w2, b2, w2g, b2g, w3, b3)


def kernel(xh0, edge_index, t, conditions, n_frag_switch, combined_mask,
           edge_attr, params):
    p = params
    feat = xh0[:, 3:]
    pos_pad = jnp.pad(xh0[:, :3], ((0, 0), (0, PD - 3)))
    t2 = t.reshape(1, 1)
    src = edge_index[0]
    dst = edge_index[1]

    w2p = jnp.pad(p['enc_W2'], ((0, 0), (0, 1)))
    b2p = jnp.pad(p['enc_b2'], (0, 1)).reshape(1, HC)
    e127 = p['emb_W'][HC - 1:HC, :]

    hcat, h16, p16 = _pre_call(feat, pos_pad, t2,
                               p['enc_W1'], p['enc_b1'].reshape(1, 256),
                               w2p, b2p,
                               p['emb_W'], e127, p['emb_b'].reshape(1, HC))

    zrows = jnp.zeros((NPT, HD), f32)
    for l in range(2):
        ew1 = p['l%d_eW1' % l]
        hs16, ps16, hd16, pd16 = _gather_call(h16, p16, src, dst)
        mw = _edge_call(hs16, ps16, hd16, pd16, edge_attr,
                        p['ee_W1'], p['ee_b1'].reshape(1, 2 * ENF),
                        p['ee_W2'], p['ee_b2'].reshape(1, ENF),
                        ew1[:HC].astype(bf16), ew1[HC:2 * HC].astype(bf16),
                        ew1[2 * HC + 1:],
                        ew1[2 * HC:2 * HC + 1],
                        p['l%d_eb1' % l].reshape(1, HC),
                        p['l%d_eW2' % l], p['l%d_eb2' % l].reshape(1, HC),
                        p['l%d_xW' % l], p['l%d_xb' % l].reshape(1, 1))
        aggf = _scatter_call(mw, dst, zrows)
        hw1 = p['l%d_hW1' % l]
        hcat, h16, p16 = _node_call(hcat, aggf.reshape(NC, N, HD),
                          hw1[:HC], hw1[HC:],
                          p['l%d_hb1' % l].reshape(1, HC),
                          p['l%d_hW2' % l], p['l%d_hb2' % l].reshape(1, HC))

    conf = _ro_call(hcat, combined_mask.reshape(N, 1),
                    p['ro_W1'], p['ro_b1'].reshape(1, HC),
                    p['ro_W1g'], p['ro_b1g'].reshape(1, HC),
                    p['ro_W2'], p['ro_b2'].reshape(1, HC),
                    p['ro_W2g'], p['ro_b2g'].reshape(1, HC),
                    p['ro_W3'], p['ro_b3'].reshape(1, 1))
    return conf.reshape(NG, 1)


# speedup vs baseline: 1.7964x; 1.7230x over previous
import functools
import jax
import jax.numpy as jnp
from jax import lax
from jax.experimental import pallas as pl
from jax.experimental.pallas import tpu as pltpu
from jax.experimental.pallas import tpu_sc as plsc

N = 10000
E = 320000
HC = 128
ENF = 16
NG = 16
PD = 16
NC = 2
NS = 16
NW = NC * NS
EPW = E // NW
CHG = 200
NITG = EPW // CHG
CH = 80
RB = 632
RBL = N - (NS - 1) * RB

BN = 2000
BE = 2000

f32 = jnp.float32


def _swish(x):
    return x * jax.nn.sigmoid(x)


def _dot(a, b):
    return jnp.dot(a, b, preferred_element_type=f32)


def _mesh():
    return plsc.VectorSubcoreMesh(core_axis_name="c", subcore_axis_name="s")


def _pre_body(feat_ref, pos_ref, t_ref, w1, b1, w2, b2, ew, e127, eb,
              h_ref, p_ref):
    z = _swish(_dot(feat_ref[...], w1[...]) + b1[...])
    hp = _dot(z, w2[...]) + b2[...]
    h_ref[...] = _dot(hp, ew[...]) + t_ref[0, 0] * e127[...] + eb[...]
    p_ref[...] = pos_ref[...]


def _pre_call(feat, pos_pad, t2, w1, b1, w2p, b2p, ew, e127, eb):
    g = N // BN
    const = lambda shape: pl.BlockSpec(shape, lambda i: (0, 0))
    return pl.pallas_call(
        _pre_body,
        grid=(g,),
        in_specs=[
            pl.BlockSpec((BN, HC), lambda i: (i, 0)),
            pl.BlockSpec((BN, PD), lambda i: (i, 0)),
            pl.BlockSpec(memory_space=pltpu.SMEM),
            const((HC, 256)), const((1, 256)),
            const((256, HC)), const((1, HC)),
            const((HC, HC)), const((1, HC)), const((1, HC)),
        ],
        out_specs=[pl.BlockSpec((BN, HC), lambda i: (i, 0)),
                   pl.BlockSpec((BN, PD), lambda i: (i, 0))],
        out_shape=[jax.ShapeDtypeStruct((N, HC), f32),
                   jax.ShapeDtypeStruct((N, PD), f32)],
    )(feat, pos_pad, t2, w1, b1, w2p, b2p, ew, e127, eb)


def _make_gather(width, tc_tiling):
    @functools.partial(
        pl.kernel,
        out_type=(jax.ShapeDtypeStruct((E, width), f32),
                  jax.ShapeDtypeStruct((E, width), f32)),
        mesh=_mesh(),
        scratch_types=(pltpu.VMEM((EPW,), jnp.int32),
                       pltpu.VMEM((2, CHG, width), f32),
                       pltpu.SemaphoreType.DMA((2,)),
                       pltpu.SemaphoreType.DMA((2,))),
        compiler_params=pltpu.CompilerParams(use_tc_tiling_on_sc=tc_tiling),
    )
    def gather_k(tab_ref, src_ref, dst_ref, osrc_ref, odst_ref,
                 idxb, buf, gsem, osem):
        wid = lax.axis_index("s") * NC + lax.axis_index("c")
        base0 = wid * EPW

        def phase(idx_hbm, out_hbm):
            pltpu.sync_copy(idx_hbm.at[pl.ds(base0, EPW)], idxb)

            def descs(k, p):
                isl = idxb.at[pl.ds(k * CHG, CHG)]
                osl = pl.ds(base0 + k * CHG, CHG)
                return (
                    pltpu.make_async_copy(tab_ref.at[isl], buf.at[p],
                                          gsem.at[p]),
                    pltpu.make_async_copy(buf.at[p], out_hbm.at[osl],
                                          osem.at[p]),
                )

            def o_wait(k):
                descs(k, k % 2)[1].wait()

            descs(0, 0)[0].start()

            def body(k, carry):
                p = k % 2

                @pl.when(k + 1 < NITG)
                def _():
                    @pl.when(k >= 1)
                    def _():
                        o_wait(k - 1)
                    descs(k + 1, 1 - p)[0].start()

                g, o = descs(k, p)
                g.wait()
                o.start()
                return carry

            lax.fori_loop(0, NITG, body, 0)
            o_wait(NITG - 2)
            o_wait(NITG - 1)

        phase(src_ref, osrc_ref)
        phase(dst_ref, odst_ref)

    return gather_k


_gather_h = _make_gather(HC, True)
_gather_p = _make_gather(PD, False)


def _tile_rows(s):
    base = s * RB
    size = jnp.where(s == NS - 1, RBL, RB)
    return base, size


def _make_scatter(width, tc_tiling):
    @functools.partial(
        pl.kernel,
        out_type=jax.ShapeDtypeStruct((NC * N, width), f32),
        mesh=_mesh(),
        scratch_types=(pltpu.VMEM((2, CH), jnp.int32),
                       pltpu.VMEM((2, CH, width), f32),
                       pltpu.SemaphoreType.DMA((2,)),
                       pltpu.SemaphoreType.DMA((2,)),
                       pltpu.VMEM_SHARED((N, width), f32)),
        compiler_params=pltpu.CompilerParams(use_tc_tiling_on_sc=tc_tiling),
    )
    def scatter_k(val_ref, dst_ref, z_ref, out_ref, ib, rb, lsem_i, lsem_r, acc):
        c = lax.axis_index("c")
        s = lax.axis_index("s")
        wid = s * NC + c
        rbase, rsize = _tile_rows(s)
        pltpu.sync_copy(z_ref.at[pl.ds(0, rsize)], acc.at[pl.ds(rbase, rsize)])
        plsc.subcore_barrier()
        base0 = wid * EPW
        nit = EPW // CH

        def l_start(k, p):
            b = base0 + k * CH
            pltpu.async_copy(dst_ref.at[pl.ds(b, CH)], ib.at[p], lsem_i.at[p])
            pltpu.async_copy(val_ref.at[pl.ds(b, CH)], rb.at[p], lsem_r.at[p])

        def l_wait(k, p):
            b = base0 + k * CH
            pltpu.make_async_copy(dst_ref.at[pl.ds(b, CH)], ib.at[p],
                                  lsem_i.at[p]).wait()
            pltpu.make_async_copy(val_ref.at[pl.ds(b, CH)], rb.at[p],
                                  lsem_r.at[p]).wait()

        l_start(0, 0)

        def body(k, carry):
            p = k % 2

            @pl.when(k + 1 < nit)
            def _():
                l_start(k + 1, 1 - p)

            l_wait(k, p)
            pltpu.sync_copy(rb.at[p], acc.at[ib.at[p]], add=True)
            return carry

        lax.fori_loop(0, nit, body, 0)
        plsc.subcore_barrier()
        pltpu.sync_copy(acc.at[pl.ds(rbase, rsize)],
                        out_ref.at[pl.ds(c * N + rbase, rsize)])

    return scatter_k


_scatter_m = _make_scatter(HC, True)
_scatter_w = _make_scatter(PD, False)


def _edge_body(hs_ref, hd_ref, ps_ref, pd_ref, ea_ref, eew1, eeb1, eew2, eeb2,
               w1s, w1d, w1e, w1d2, b1, w2, b2, xw, xb, m_ref, wr_ref):
    rel = ps_ref[...] - pd_ref[...]
    d2 = jnp.sum(rel * rel, axis=1, keepdims=True)
    e = _dot(_swish(_dot(ea_ref[...], eew1[...]) + eeb1[...]), eew2[...]) + eeb2[...]
    pre = (_dot(hs_ref[...], w1s[...]) + _dot(hd_ref[...], w1d[...])
           + _dot(e, w1e[...]) + d2 * w1d2[...] + b1[...])
    m = _swish(_dot(_swish(pre), w2[...]) + b2[...])
    coef = _dot(m, xw[...]) + xb[...]
    m_ref[...] = m
    wr_ref[...] = rel * (coef / (jnp.sqrt(d2) + 1.0))


def _edge_call(hs, hd, ps, pd, edge_attr, eew1, eeb1, eew2, eeb2,
               w1s, w1d, w1e, w1d2, b1, w2, b2, xw, xb):
    g = E // BE
    const = lambda shape: pl.BlockSpec(shape, lambda i: (0, 0))
    return pl.pallas_call(
        _edge_body,
        grid=(g,),
        in_specs=[
            pl.BlockSpec((BE, HC), lambda i: (i, 0)),
            pl.BlockSpec((BE, HC), lambda i: (i, 0)),
            pl.BlockSpec((BE, PD), lambda i: (i, 0)),
            pl.BlockSpec((BE, PD), lambda i: (i, 0)),
            pl.BlockSpec((BE, ENF), lambda i: (i, 0)),
            const((ENF, 2 * ENF)), const((1, 2 * ENF)),
            const((2 * ENF, ENF)), const((1, ENF)),
            const((HC, HC)), const((HC, HC)), const((ENF, HC)),
            const((1, HC)), const((1, HC)),
            const((HC, HC)), const((1, HC)),
            const((HC, 1)), const((1, 1)),
        ],
        out_specs=[pl.BlockSpec((BE, HC), lambda i: (i, 0)),
                   pl.BlockSpec((BE, PD), lambda i: (i, 0))],
        out_shape=[jax.ShapeDtypeStruct((E, HC), f32),
                   jax.ShapeDtypeStruct((E, PD), f32)],
    )(hs, hd, ps, pd, edge_attr, eew1, eeb1, eew2, eeb2,
      w1s, w1d, w1e, w1d2, b1, w2, b2, xw, xb)


def _node_body(h_ref, p_ref, agg_ref, pd_ref, w1h, w1a, b1, w2, b2,
               ho_ref, po_ref):
    h = h_ref[...]
    ag = agg_ref[0] + agg_ref[1]
    u = _swish(_dot(h, w1h[...]) + _dot(ag, w1a[...]) + b1[...])
    ho_ref[...] = h + _dot(u, w2[...]) + b2[...]
    po_ref[...] = p_ref[...] + pd_ref[0] + pd_ref[1]


def _node_call(h32, pos_t, aggp, posdp, w1h, w1a, b1, w2, b2):
    g = N // BN
    const = lambda shape: pl.BlockSpec(shape, lambda i: (0, 0))
    return pl.pallas_call(
        _node_body,
        grid=(g,),
        in_specs=[
            pl.BlockSpec((BN, HC), lambda i: (i, 0)),
            pl.BlockSpec((BN, PD), lambda i: (i, 0)),
            pl.BlockSpec((NC, BN, HC), lambda i: (0, i, 0)),
            pl.BlockSpec((NC, BN, PD), lambda i: (0, i, 0)),
            const((HC, HC)), const((HC, HC)), const((1, HC)),
            const((HC, HC)), const((1, HC)),
        ],
        out_specs=[pl.BlockSpec((BN, HC), lambda i: (i, 0)),
                   pl.BlockSpec((BN, PD), lambda i: (i, 0))],
        out_shape=[jax.ShapeDtypeStruct((N, HC), f32),
                   jax.ShapeDtypeStruct((N, PD), f32)],
    )(h32, pos_t, aggp, posdp, w1h, w1a, b1, w2, b2)


def _ro_body(h_ref, mk_ref, w1, b1, w1g, b1g, w2, b2, w2g, b2g, w3, b3,
             conf_ref, s_acc, c_acc):
    i = pl.program_id(0)

    @pl.when(i == 0)
    def _():
        s_acc[...] = jnp.zeros_like(s_acc)
        c_acc[...] = jnp.zeros_like(c_acc)

    h = h_ref[...]
    g1 = jax.nn.sigmoid(_dot(h, w1g[...]) + b1g[...])
    v = _swish((_dot(h, w1[...]) + b1[...]) * g1)
    g2 = jax.nn.sigmoid(_dot(v, w2g[...]) + b2g[...])
    v = _swish((_dot(v, w2[...]) + b2[...]) * g2)
    nout = _dot(v, w3[...]) + b3[...]
    oh = (mk_ref[...] == lax.broadcasted_iota(jnp.int32, (BN, NG), 1)).astype(f32)
    s_acc[...] += jnp.sum(oh * nout, axis=0, keepdims=True)
    c_acc[...] += jnp.sum(oh, axis=0, keepdims=True)
    conf_ref[...] = s_acc[...] / jnp.maximum(c_acc[...], 1.0)


def _ro_call(h32, mask2, w1, b1, w1g, b1g, w2, b2, w2g, b2g, w3, b3):
    g = N // BN
    const = lambda shape: pl.BlockSpec(shape, lambda i: (0, 0))
    return pl.pallas_call(
        _ro_body,
        grid=(g,),
        in_specs=[
            pl.BlockSpec((BN, HC), lambda i: (i, 0)),
            pl.BlockSpec((BN, 1), lambda i: (i, 0)),
            const((HC, HC)), const((1, HC)),
            const((HC, HC)), const((1, HC)),
            const((HC, HC)), const((1, HC)),
            const((HC, HC)), const((1, HC)),
            const((HC, 1)), const((1, 1)),
        ],
        out_specs=pl.BlockSpec((1, NG), lambda i: (0, 0)),
        out_shape=jax.ShapeDtypeStruct((1, NG), f32),
        scratch_shapes=[pltpu.VMEM((1, NG), f32), pltpu.VMEM((1, NG), f32)],
    )(h32, mask2, w1, b1, w1g, b1g, w2, b2, w2g, b2g, w3, b3)


def kernel(xh0, edge_index, t, conditions, n_frag_switch, combined_mask,
           edge_attr, params):
    p = params
    feat = xh0[:, 3:]
    pos_pad = jnp.pad(xh0[:, :3], ((0, 0), (0, PD - 3)))
    t2 = t.reshape(1, 1)
    src = edge_index[0]
    dst = edge_index[1]

    w2p = jnp.pad(p['enc_W2'], ((0, 0), (0, 1)))
    b2p = jnp.pad(p['enc_b2'], (0, 1)).reshape(1, HC)
    e127 = p['emb_W'][HC - 1:HC, :]

    h32, pos_t = _pre_call(feat, pos_pad, t2,
                           p['enc_W1'], p['enc_b1'].reshape(1, 256),
                           w2p, b2p,
                           p['emb_W'], e127, p['emb_b'].reshape(1, HC))

    zrows = jnp.zeros((RB, HC), f32)
    zrows_p = jnp.zeros((RB, PD), f32)
    for l in range(2):
        ew1 = p['l%d_eW1' % l]
        hs, hd = _gather_h(h32, src, dst)
        ps, pd_ = _gather_p(pos_t, src, dst)
        m, wr = _edge_call(hs, hd, ps, pd_, edge_attr,
                           p['ee_W1'], p['ee_b1'].reshape(1, 2 * ENF),
                           p['ee_W2'], p['ee_b2'].reshape(1, ENF),
                           ew1[:HC], ew1[HC:2 * HC], ew1[2 * HC + 1:],
                           ew1[2 * HC:2 * HC + 1],
                           p['l%d_eb1' % l].reshape(1, HC),
                           p['l%d_eW2' % l], p['l%d_eb2' % l].reshape(1, HC),
                           p['l%d_xW' % l], p['l%d_xb' % l].reshape(1, 1))
        aggf = _scatter_m(m, dst, zrows)
        posdf = _scatter_w(wr, dst, zrows_p)
        hw1 = p['l%d_hW1' % l]
        h32, pos_t = _node_call(h32, pos_t,
                                aggf.reshape(NC, N, HC),
                                posdf.reshape(NC, N, PD),
                                hw1[:HC], hw1[HC:],
                                p['l%d_hb1' % l].reshape(1, HC),
                                p['l%d_hW2' % l],
                                p['l%d_hb2' % l].reshape(1, HC))

    conf = _ro_call(h32, combined_mask.reshape(N, 1),
                    p['ro_W1'], p['ro_b1'].reshape(1, HC),
                    p['ro_W1g'], p['ro_b1g'].reshape(1, HC),
                    p['ro_W2'], p['ro_b2'].reshape(1, HC),
                    p['ro_W2g'], p['ro_b2g'].reshape(1, HC),
                    p['ro_W3'], p['ro_b3'].reshape(1, 1))
    return conf.reshape(NG, 1)


# speedup vs baseline: 2.0985x; 1.1682x over previous
import functools
import jax
import jax.numpy as jnp
from jax import lax
from jax.experimental import pallas as pl
from jax.experimental.pallas import tpu as pltpu
from jax.experimental.pallas import tpu_sc as plsc

N = 10000
E = 320000
HC = 128
ENF = 16
NG = 16
PD = 16
NC = 2
NS = 16
NW = NC * NS
EPW = E // NW
CHG = 200
NITG = EPW // CHG
CH = 80
RB = 632
RBL = N - (NS - 1) * RB

BN = 2000
BE = 2560

f32 = jnp.float32


def _swish(x):
    return x * jax.nn.sigmoid(x)


def _dot(a, b):
    return jnp.dot(a, b, preferred_element_type=f32)


def _mesh():
    return plsc.VectorSubcoreMesh(core_axis_name="c", subcore_axis_name="s")


def _pre_body(feat_ref, pos_ref, t_ref, w1, b1, w2, b2, ew, e127, eb,
              h_ref, p_ref):
    z = _swish(_dot(feat_ref[...], w1[...]) + b1[...])
    hp = _dot(z, w2[...]) + b2[...]
    h_ref[...] = _dot(hp, ew[...]) + t_ref[0, 0] * e127[...] + eb[...]
    p_ref[...] = pos_ref[...]


def _pre_call(feat, pos_pad, t2, w1, b1, w2p, b2p, ew, e127, eb):
    g = N // BN
    const = lambda shape: pl.BlockSpec(shape, lambda i: (0, 0))
    return pl.pallas_call(
        _pre_body,
        grid=(g,),
        in_specs=[
            pl.BlockSpec((BN, HC), lambda i: (i, 0)),
            pl.BlockSpec((BN, PD), lambda i: (i, 0)),
            pl.BlockSpec(memory_space=pltpu.SMEM),
            const((HC, 256)), const((1, 256)),
            const((256, HC)), const((1, HC)),
            const((HC, HC)), const((1, HC)), const((1, HC)),
        ],
        out_specs=[pl.BlockSpec((BN, HC), lambda i: (i, 0)),
                   pl.BlockSpec((BN, PD), lambda i: (i, 0))],
        out_shape=[jax.ShapeDtypeStruct((N, HC), f32),
                   jax.ShapeDtypeStruct((N, PD), f32)],
    )(feat, pos_pad, t2, w1, b1, w2p, b2p, ew, e127, eb)


def _make_gather(width, owidth, tc_tiling):
    @functools.partial(
        pl.kernel,
        out_type=(jax.ShapeDtypeStruct((E, owidth), f32),
                  jax.ShapeDtypeStruct((E, owidth), f32)),
        mesh=_mesh(),
        scratch_types=(pltpu.VMEM((EPW,), jnp.int32),
                       pltpu.VMEM((2, CHG, width), f32),
                       pltpu.SemaphoreType.DMA((2,)),
                       pltpu.SemaphoreType.DMA((2,))),
        compiler_params=pltpu.CompilerParams(use_tc_tiling_on_sc=tc_tiling),
    )
    def gather_k(tab_ref, src_ref, dst_ref, osrc_ref, odst_ref,
                 idxb, buf, gsem, osem):
        wid = lax.axis_index("s") * NC + lax.axis_index("c")
        base0 = wid * EPW

        def phase(idx_hbm, out_hbm):
            pltpu.sync_copy(idx_hbm.at[pl.ds(base0, EPW)], idxb)

            def descs(k, p):
                isl = idxb.at[pl.ds(k * CHG, CHG)]
                if owidth == width:
                    odst = out_hbm.at[pl.ds(base0 + k * CHG, CHG)]
                else:
                    odst = out_hbm.at[pl.ds(base0 + k * CHG, CHG),
                                      pl.ds(0, width)]
                return (
                    pltpu.make_async_copy(tab_ref.at[isl], buf.at[p],
                                          gsem.at[p]),
                    pltpu.make_async_copy(buf.at[p], odst, osem.at[p]),
                )

            def o_wait(k):
                descs(k, k % 2)[1].wait()

            descs(0, 0)[0].start()

            def body(k, carry):
                p = k % 2

                @pl.when(k + 1 < NITG)
                def _():
                    @pl.when(k >= 1)
                    def _():
                        o_wait(k - 1)
                    descs(k + 1, 1 - p)[0].start()

                g, o = descs(k, p)
                g.wait()
                o.start()
                return carry

            lax.fori_loop(0, NITG, body, 0)
            o_wait(NITG - 2)
            o_wait(NITG - 1)

        phase(src_ref, osrc_ref)
        phase(dst_ref, odst_ref)

    return gather_k


_gather_h = _make_gather(HC, HC, True)
_gather_p = _make_gather(PD, HC, False)


def _tile_rows(s):
    base = s * RB
    size = jnp.where(s == NS - 1, RBL, RB)
    return base, size


def _make_scatter(width, tc_tiling):
    @functools.partial(
        pl.kernel,
        out_type=jax.ShapeDtypeStruct((NC * N, width), f32),
        mesh=_mesh(),
        scratch_types=(pltpu.VMEM((2, CH), jnp.int32),
                       pltpu.VMEM((2, CH, width), f32),
                       pltpu.SemaphoreType.DMA((2,)),
                       pltpu.SemaphoreType.DMA((2,)),
                       pltpu.VMEM_SHARED((N, width), f32)),
        compiler_params=pltpu.CompilerParams(use_tc_tiling_on_sc=tc_tiling),
    )
    def scatter_k(val_ref, dst_ref, z_ref, out_ref, ib, rb, lsem_i, lsem_r, acc):
        c = lax.axis_index("c")
        s = lax.axis_index("s")
        wid = s * NC + c
        rbase, rsize = _tile_rows(s)
        pltpu.sync_copy(z_ref.at[pl.ds(0, rsize)], acc.at[pl.ds(rbase, rsize)])
        plsc.subcore_barrier()
        base0 = wid * EPW
        nit = EPW // CH

        def l_start(k, p):
            b = base0 + k * CH
            pltpu.async_copy(dst_ref.at[pl.ds(b, CH)], ib.at[p], lsem_i.at[p])
            pltpu.async_copy(val_ref.at[pl.ds(b, CH)], rb.at[p], lsem_r.at[p])

        def l_wait(k, p):
            b = base0 + k * CH
            pltpu.make_async_copy(dst_ref.at[pl.ds(b, CH)], ib.at[p],
                                  lsem_i.at[p]).wait()
            pltpu.make_async_copy(val_ref.at[pl.ds(b, CH)], rb.at[p],
                                  lsem_r.at[p]).wait()

        l_start(0, 0)

        def body(k, carry):
            p = k % 2

            @pl.when(k + 1 < nit)
            def _():
                l_start(k + 1, 1 - p)

            l_wait(k, p)
            pltpu.sync_copy(rb.at[p], acc.at[ib.at[p]], add=True)
            return carry

        lax.fori_loop(0, nit, body, 0)
        plsc.subcore_barrier()
        pltpu.sync_copy(acc.at[pl.ds(rbase, rsize)],
                        out_ref.at[pl.ds(c * N + rbase, rsize)])

    return scatter_k


_scatter_m = _make_scatter(HC, True)


def _edge_body(hs_ref, hd_ref, ps_ref, pd_ref, ea_ref, eew1, eeb1, eew2, eeb2,
               w1s, w1d, w1e, w1d2, b1, w2, b2, xw, xb, m_ref, wr_ref):
    rel = ps_ref[:, :PD] - pd_ref[:, :PD]
    d2 = jnp.sum(rel * rel, axis=1, keepdims=True)
    e = _dot(_swish(_dot(ea_ref[...], eew1[...]) + eeb1[...]), eew2[...]) + eeb2[...]
    pre = (_dot(hs_ref[...], w1s[...]) + _dot(hd_ref[...], w1d[...])
           + _dot(e, w1e[...]) + d2 * w1d2[...] + b1[...])
    m = _swish(_dot(_swish(pre), w2[...]) + b2[...])
    coef = _dot(m, xw[...]) + xb[...]
    m_ref[...] = m
    wr_ref[:, :PD] = rel * (coef / (jnp.sqrt(d2) + 1.0))
    wr_ref[:, PD:] = jnp.zeros((wr_ref.shape[0], HC - PD), f32)


def _edge_call(hs, hd, ps, pd, edge_attr, eew1, eeb1, eew2, eeb2,
               w1s, w1d, w1e, w1d2, b1, w2, b2, xw, xb):
    g = E // BE
    const = lambda shape: pl.BlockSpec(shape, lambda i: (0, 0))
    return pl.pallas_call(
        _edge_body,
        grid=(g,),
        in_specs=[
            pl.BlockSpec((BE, HC), lambda i: (i, 0)),
            pl.BlockSpec((BE, HC), lambda i: (i, 0)),
            pl.BlockSpec((BE, HC), lambda i: (i, 0)),
            pl.BlockSpec((BE, HC), lambda i: (i, 0)),
            pl.BlockSpec((BE, ENF), lambda i: (i, 0)),
            const((ENF, 2 * ENF)), const((1, 2 * ENF)),
            const((2 * ENF, ENF)), const((1, ENF)),
            const((HC, HC)), const((HC, HC)), const((ENF, HC)),
            const((1, HC)), const((1, HC)),
            const((HC, HC)), const((1, HC)),
            const((HC, 1)), const((1, 1)),
        ],
        out_specs=[pl.BlockSpec((BE, HC), lambda i: (i, 0)),
                   pl.BlockSpec((BE, HC), lambda i: (i, 0))],
        out_shape=[jax.ShapeDtypeStruct((E, HC), f32),
                   jax.ShapeDtypeStruct((E, HC), f32)],
    )(hs, hd, ps, pd, edge_attr, eew1, eeb1, eew2, eeb2,
      w1s, w1d, w1e, w1d2, b1, w2, b2, xw, xb)


def _node_body(h_ref, p_ref, agg_ref, pd_ref, w1h, w1a, b1, w2, b2,
               ho_ref, po_ref):
    h = h_ref[...]
    ag = agg_ref[0] + agg_ref[1]
    u = _swish(_dot(h, w1h[...]) + _dot(ag, w1a[...]) + b1[...])
    ho_ref[...] = h + _dot(u, w2[...]) + b2[...]
    po_ref[...] = p_ref[...] + pd_ref[0][:, :PD] + pd_ref[1][:, :PD]


def _node_call(h32, pos_t, aggp, posdp, w1h, w1a, b1, w2, b2):
    g = N // BN
    const = lambda shape: pl.BlockSpec(shape, lambda i: (0, 0))
    return pl.pallas_call(
        _node_body,
        grid=(g,),
        in_specs=[
            pl.BlockSpec((BN, HC), lambda i: (i, 0)),
            pl.BlockSpec((BN, PD), lambda i: (i, 0)),
            pl.BlockSpec((NC, BN, HC), lambda i: (0, i, 0)),
            pl.BlockSpec((NC, BN, HC), lambda i: (0, i, 0)),
            const((HC, HC)), const((HC, HC)), const((1, HC)),
            const((HC, HC)), const((1, HC)),
        ],
        out_specs=[pl.BlockSpec((BN, HC), lambda i: (i, 0)),
                   pl.BlockSpec((BN, PD), lambda i: (i, 0))],
        out_shape=[jax.ShapeDtypeStruct((N, HC), f32),
                   jax.ShapeDtypeStruct((N, PD), f32)],
    )(h32, pos_t, aggp, posdp, w1h, w1a, b1, w2, b2)


def _ro_body(h_ref, mk_ref, w1, b1, w1g, b1g, w2, b2, w2g, b2g, w3, b3,
             conf_ref, s_acc, c_acc):
    i = pl.program_id(0)

    @pl.when(i == 0)
    def _():
        s_acc[...] = jnp.zeros_like(s_acc)
        c_acc[...] = jnp.zeros_like(c_acc)

    h = h_ref[...]
    g1 = jax.nn.sigmoid(_dot(h, w1g[...]) + b1g[...])
    v = _swish((_dot(h, w1[...]) + b1[...]) * g1)
    g2 = jax.nn.sigmoid(_dot(v, w2g[...]) + b2g[...])
    v = _swish((_dot(v, w2[...]) + b2[...]) * g2)
    nout = _dot(v, w3[...]) + b3[...]
    oh = (mk_ref[...] == lax.broadcasted_iota(jnp.int32, (BN, NG), 1)).astype(f32)
    s_acc[...] += jnp.sum(oh * nout, axis=0, keepdims=True)
    c_acc[...] += jnp.sum(oh, axis=0, keepdims=True)
    conf_ref[...] = s_acc[...] / jnp.maximum(c_acc[...], 1.0)


def _ro_call(h32, mask2, w1, b1, w1g, b1g, w2, b2, w2g, b2g, w3, b3):
    g = N // BN
    const = lambda shape: pl.BlockSpec(shape, lambda i: (0, 0))
    return pl.pallas_call(
        _ro_body,
        grid=(g,),
        in_specs=[
            pl.BlockSpec((BN, HC), lambda i: (i, 0)),
            pl.BlockSpec((BN, 1), lambda i: (i, 0)),
            const((HC, HC)), const((1, HC)),
            const((HC, HC)), const((1, HC)),
            const((HC, HC)), const((1, HC)),
            const((HC, HC)), const((1, HC)),
            const((HC, 1)), const((1, 1)),
        ],
        out_specs=pl.BlockSpec((1, NG), lambda i: (0, 0)),
        out_shape=jax.ShapeDtypeStruct((1, NG), f32),
        scratch_shapes=[pltpu.VMEM((1, NG), f32), pltpu.VMEM((1, NG), f32)],
    )(h32, mask2, w1, b1, w1g, b1g, w2, b2, w2g, b2g, w3, b3)


def kernel(xh0, edge_index, t, conditions, n_frag_switch, combined_mask,
           edge_attr, params):
    p = params
    feat = xh0[:, 3:]
    pos_pad = jnp.pad(xh0[:, :3], ((0, 0), (0, PD - 3)))
    t2 = t.reshape(1, 1)
    src = edge_index[0]
    dst = edge_index[1]

    w2p = jnp.pad(p['enc_W2'], ((0, 0), (0, 1)))
    b2p = jnp.pad(p['enc_b2'], (0, 1)).reshape(1, HC)
    e127 = p['emb_W'][HC - 1:HC, :]

    h32, pos_t = _pre_call(feat, pos_pad, t2,
                           p['enc_W1'], p['enc_b1'].reshape(1, 256),
                           w2p, b2p,
                           p['emb_W'], e127, p['emb_b'].reshape(1, HC))

    zrows = jnp.zeros((RB, HC), f32)
    for l in range(2):
        ew1 = p['l%d_eW1' % l]
        hs, hd = _gather_h(h32, src, dst)
        ps, pd_ = _gather_p(pos_t, src, dst)
        m, wr = _edge_call(hs, hd, ps, pd_, edge_attr,
                           p['ee_W1'], p['ee_b1'].reshape(1, 2 * ENF),
                           p['ee_W2'], p['ee_b2'].reshape(1, ENF),
                           ew1[:HC], ew1[HC:2 * HC], ew1[2 * HC + 1:],
                           ew1[2 * HC:2 * HC + 1],
                           p['l%d_eb1' % l].reshape(1, HC),
                           p['l%d_eW2' % l], p['l%d_eb2' % l].reshape(1, HC),
                           p['l%d_xW' % l], p['l%d_xb' % l].reshape(1, 1))
        aggf = _scatter_m(m, dst, zrows)
        posdf = _scatter_m(wr, dst, zrows)
        hw1 = p['l%d_hW1' % l]
        h32, pos_t = _node_call(h32, pos_t,
                                aggf.reshape(NC, N, HC),
                                posdf.reshape(NC, N, HC),
                                hw1[:HC], hw1[HC:],
                                p['l%d_hb1' % l].reshape(1, HC),
                                p['l%d_hW2' % l],
                                p['l%d_hb2' % l].reshape(1, HC))

    conf = _ro_call(h32, combined_mask.reshape(N, 1),
                    p['ro_W1'], p['ro_b1'].reshape(1, HC),
                    p['ro_W1g'], p['ro_b1g'].reshape(1, HC),
                    p['ro_W2'], p['ro_b2'].reshape(1, HC),
                    p['ro_W2g'], p['ro_b2g'].reshape(1, HC),
                    p['ro_W3'], p['ro_b3'].reshape(1, 1))
    return conf.reshape(NG, 1)


# speedup vs baseline: 2.1602x; 1.0294x over previous
import functools
import jax
import jax.numpy as jnp
from jax import lax
from jax.experimental import pallas as pl
from jax.experimental.pallas import tpu as pltpu
from jax.experimental.pallas import tpu_sc as plsc

N = 10000
E = 320000
HC = 128
ENF = 16
NG = 16
PD = 16
NC = 2
NS = 16
NW = NC * NS
EPW = E // NW
CHG = 200
NITG = EPW // CHG
CH = 80
RB = 632
RBL = N - (NS - 1) * RB

BN = 2000
BE = 2560

f32 = jnp.float32


def _swish(x):
    return x * jax.nn.sigmoid(x)


def _dot(a, b):
    return jnp.dot(a, b, preferred_element_type=f32)


def _mesh():
    return plsc.VectorSubcoreMesh(core_axis_name="c", subcore_axis_name="s")


def _pre_body(feat_ref, pos_ref, t_ref, w1, b1, w2, b2, ew, e127, eb,
              h_ref, p_ref):
    z = _swish(_dot(feat_ref[...], w1[...]) + b1[...])
    hp = _dot(z, w2[...]) + b2[...]
    h_ref[...] = _dot(hp, ew[...]) + t_ref[0, 0] * e127[...] + eb[...]
    p_ref[...] = pos_ref[...]


def _pre_call(feat, pos_pad, t2, w1, b1, w2p, b2p, ew, e127, eb):
    g = N // BN
    const = lambda shape: pl.BlockSpec(shape, lambda i: (0, 0))
    return pl.pallas_call(
        _pre_body,
        grid=(g,),
        in_specs=[
            pl.BlockSpec((BN, HC), lambda i: (i, 0)),
            pl.BlockSpec((BN, PD), lambda i: (i, 0)),
            pl.BlockSpec(memory_space=pltpu.SMEM),
            const((HC, 256)), const((1, 256)),
            const((256, HC)), const((1, HC)),
            const((HC, HC)), const((1, HC)), const((1, HC)),
        ],
        out_specs=[pl.BlockSpec((BN, HC), lambda i: (i, 0)),
                   pl.BlockSpec((BN, PD), lambda i: (i, 0))],
        out_shape=[jax.ShapeDtypeStruct((N, HC), f32),
                   jax.ShapeDtypeStruct((N, PD), f32)],
    )(feat, pos_pad, t2, w1, b1, w2p, b2p, ew, e127, eb)


@functools.partial(
    pl.kernel,
    out_type=(jax.ShapeDtypeStruct((E, HC), f32),
              jax.ShapeDtypeStruct((E, HC), f32),
              jax.ShapeDtypeStruct((E, HC), f32),
              jax.ShapeDtypeStruct((E, HC), f32)),
    mesh=_mesh(),
    scratch_types=(pltpu.VMEM((EPW,), jnp.int32),
                   pltpu.VMEM((2, CHG, HC), f32),
                   pltpu.VMEM((2, CHG, PD), f32),
                   pltpu.SemaphoreType.DMA((2,)),
                   pltpu.SemaphoreType.DMA((2,))),
    compiler_params=pltpu.CompilerParams(use_tc_tiling_on_sc=False),
)
def _gather_hp(h_ref, p_ref, src_ref, dst_ref,
               ohs_ref, ohd_ref, ops_ref, opd_ref,
               idxb, bufh, bufp, gsem, osem):
    wid = lax.axis_index("s") * NC + lax.axis_index("c")
    base0 = wid * EPW

    def phase(idx_hbm, oh_hbm, op_hbm):
        pltpu.sync_copy(idx_hbm.at[pl.ds(base0, EPW)], idxb)

        def descs(k, p):
            isl = idxb.at[pl.ds(k * CHG, CHG)]
            rsl = pl.ds(base0 + k * CHG, CHG)
            return (
                pltpu.make_async_copy(h_ref.at[isl], bufh.at[p], gsem.at[p]),
                pltpu.make_async_copy(p_ref.at[isl], bufp.at[p], gsem.at[p]),
                pltpu.make_async_copy(bufh.at[p], oh_hbm.at[rsl], osem.at[p]),
                pltpu.make_async_copy(bufp.at[p],
                                      op_hbm.at[rsl, pl.ds(0, PD)],
                                      osem.at[p]),
            )

        def g_start(k, p):
            gh, gp, _, _ = descs(k, p)
            gh.start()
            gp.start()

        def o_wait(k):
            _, _, oh, op = descs(k, k % 2)
            oh.wait()
            op.wait()

        g_start(0, 0)

        def body(k, carry):
            p = k % 2

            @pl.when(k + 1 < NITG)
            def _():
                @pl.when(k >= 1)
                def _():
                    o_wait(k - 1)
                g_start(k + 1, 1 - p)

            gh, gp, oh, op = descs(k, p)
            gh.wait()
            gp.wait()
            oh.start()
            op.start()
            return carry

        lax.fori_loop(0, NITG, body, 0)
        o_wait(NITG - 2)
        o_wait(NITG - 1)

    phase(src_ref, ohs_ref, ops_ref)
    phase(dst_ref, ohd_ref, opd_ref)


def _tile_rows(s):
    base = s * RB
    size = jnp.where(s == NS - 1, RBL, RB)
    return base, size


def _make_scatter(width, tc_tiling):
    @functools.partial(
        pl.kernel,
        out_type=jax.ShapeDtypeStruct((NC * N, width), f32),
        mesh=_mesh(),
        scratch_types=(pltpu.VMEM((2, CH), jnp.int32),
                       pltpu.VMEM((2, CH, width), f32),
                       pltpu.SemaphoreType.DMA((2,)),
                       pltpu.SemaphoreType.DMA((2,)),
                       pltpu.VMEM_SHARED((N, width), f32)),
        compiler_params=pltpu.CompilerParams(use_tc_tiling_on_sc=tc_tiling),
    )
    def scatter_k(val_ref, dst_ref, z_ref, out_ref, ib, rb, lsem_i, lsem_r, acc):
        c = lax.axis_index("c")
        s = lax.axis_index("s")
        wid = s * NC + c
        rbase, rsize = _tile_rows(s)
        pltpu.sync_copy(z_ref.at[pl.ds(0, rsize)], acc.at[pl.ds(rbase, rsize)])
        plsc.subcore_barrier()
        base0 = wid * EPW
        nit = EPW // CH

        def l_start(k, p):
            b = base0 + k * CH
            pltpu.async_copy(dst_ref.at[pl.ds(b, CH)], ib.at[p], lsem_i.at[p])
            pltpu.async_copy(val_ref.at[pl.ds(b, CH)], rb.at[p], lsem_r.at[p])

        def l_wait(k, p):
            b = base0 + k * CH
            pltpu.make_async_copy(dst_ref.at[pl.ds(b, CH)], ib.at[p],
                                  lsem_i.at[p]).wait()
            pltpu.make_async_copy(val_ref.at[pl.ds(b, CH)], rb.at[p],
                                  lsem_r.at[p]).wait()

        l_start(0, 0)

        def body(k, carry):
            p = k % 2

            @pl.when(k + 1 < nit)
            def _():
                l_start(k + 1, 1 - p)

            l_wait(k, p)
            pltpu.sync_copy(rb.at[p], acc.at[ib.at[p]], add=True)
            return carry

        lax.fori_loop(0, nit, body, 0)
        plsc.subcore_barrier()
        pltpu.sync_copy(acc.at[pl.ds(rbase, rsize)],
                        out_ref.at[pl.ds(c * N + rbase, rsize)])

    return scatter_k


_scatter_m = _make_scatter(HC, True)


def _edge_body(hs_ref, hd_ref, ps_ref, pd_ref, ea_ref, eew1, eeb1, eew2, eeb2,
               w1s, w1d, w1e, w1d2, b1, w2, b2, xw, xb, m_ref, wr_ref):
    rel = ps_ref[:, :PD] - pd_ref[:, :PD]
    d2 = jnp.sum(rel * rel, axis=1, keepdims=True)
    e = _dot(_swish(_dot(ea_ref[...], eew1[...]) + eeb1[...]), eew2[...]) + eeb2[...]
    pre = (_dot(hs_ref[...], w1s[...]) + _dot(hd_ref[...], w1d[...])
           + _dot(e, w1e[...]) + d2 * w1d2[...] + b1[...])
    m = _swish(_dot(_swish(pre), w2[...]) + b2[...])
    coef = _dot(m, xw[...]) + xb[...]
    m_ref[...] = m
    wr_ref[:, :PD] = rel * (coef / (jnp.sqrt(d2) + 1.0))
    wr_ref[:, PD:] = jnp.zeros((wr_ref.shape[0], HC - PD), f32)


def _edge_call(hs, hd, ps, pd, edge_attr, eew1, eeb1, eew2, eeb2,
               w1s, w1d, w1e, w1d2, b1, w2, b2, xw, xb):
    g = E // BE
    const = lambda shape: pl.BlockSpec(shape, lambda i: (0, 0))
    return pl.pallas_call(
        _edge_body,
        grid=(g,),
        in_specs=[
            pl.BlockSpec((BE, HC), lambda i: (i, 0)),
            pl.BlockSpec((BE, HC), lambda i: (i, 0)),
            pl.BlockSpec((BE, HC), lambda i: (i, 0)),
            pl.BlockSpec((BE, HC), lambda i: (i, 0)),
            pl.BlockSpec((BE, ENF), lambda i: (i, 0)),
            const((ENF, 2 * ENF)), const((1, 2 * ENF)),
            const((2 * ENF, ENF)), const((1, ENF)),
            const((HC, HC)), const((HC, HC)), const((ENF, HC)),
            const((1, HC)), const((1, HC)),
            const((HC, HC)), const((1, HC)),
            const((HC, 1)), const((1, 1)),
        ],
        out_specs=[pl.BlockSpec((BE, HC), lambda i: (i, 0)),
                   pl.BlockSpec((BE, HC), lambda i: (i, 0))],
        out_shape=[jax.ShapeDtypeStruct((E, HC), f32),
                   jax.ShapeDtypeStruct((E, HC), f32)],
    )(hs, hd, ps, pd, edge_attr, eew1, eeb1, eew2, eeb2,
      w1s, w1d, w1e, w1d2, b1, w2, b2, xw, xb)


def _node_body(h_ref, p_ref, agg_ref, pd_ref, w1h, w1a, b1, w2, b2,
               ho_ref, po_ref):
    h = h_ref[...]
    ag = agg_ref[0] + agg_ref[1]
    u = _swish(_dot(h, w1h[...]) + _dot(ag, w1a[...]) + b1[...])
    ho_ref[...] = h + _dot(u, w2[...]) + b2[...]
    po_ref[...] = p_ref[...] + pd_ref[0][:, :PD] + pd_ref[1][:, :PD]


def _node_call(h32, pos_t, aggp, posdp, w1h, w1a, b1, w2, b2):
    g = N // BN
    const = lambda shape: pl.BlockSpec(shape, lambda i: (0, 0))
    return pl.pallas_call(
        _node_body,
        grid=(g,),
        in_specs=[
            pl.BlockSpec((BN, HC), lambda i: (i, 0)),
            pl.BlockSpec((BN, PD), lambda i: (i, 0)),
            pl.BlockSpec((NC, BN, HC), lambda i: (0, i, 0)),
            pl.BlockSpec((NC, BN, HC), lambda i: (0, i, 0)),
            const((HC, HC)), const((HC, HC)), const((1, HC)),
            const((HC, HC)), const((1, HC)),
        ],
        out_specs=[pl.BlockSpec((BN, HC), lambda i: (i, 0)),
                   pl.BlockSpec((BN, PD), lambda i: (i, 0))],
        out_shape=[jax.ShapeDtypeStruct((N, HC), f32),
                   jax.ShapeDtypeStruct((N, PD), f32)],
    )(h32, pos_t, aggp, posdp, w1h, w1a, b1, w2, b2)


def _ro_body(h_ref, mk_ref, w1, b1, w1g, b1g, w2, b2, w2g, b2g, w3, b3,
             conf_ref, s_acc, c_acc):
    i = pl.program_id(0)

    @pl.when(i == 0)
    def _():
        s_acc[...] = jnp.zeros_like(s_acc)
        c_acc[...] = jnp.zeros_like(c_acc)

    h = h_ref[...]
    g1 = jax.nn.sigmoid(_dot(h, w1g[...]) + b1g[...])
    v = _swish((_dot(h, w1[...]) + b1[...]) * g1)
    g2 = jax.nn.sigmoid(_dot(v, w2g[...]) + b2g[...])
    v = _swish((_dot(v, w2[...]) + b2[...]) * g2)
    nout = _dot(v, w3[...]) + b3[...]
    oh = (mk_ref[...] == lax.broadcasted_iota(jnp.int32, (BN, NG), 1)).astype(f32)
    s_acc[...] += jnp.sum(oh * nout, axis=0, keepdims=True)
    c_acc[...] += jnp.sum(oh, axis=0, keepdims=True)
    conf_ref[...] = s_acc[...] / jnp.maximum(c_acc[...], 1.0)


def _ro_call(h32, mask2, w1, b1, w1g, b1g, w2, b2, w2g, b2g, w3, b3):
    g = N // BN
    const = lambda shape: pl.BlockSpec(shape, lambda i: (0, 0))
    return pl.pallas_call(
        _ro_body,
        grid=(g,),
        in_specs=[
            pl.BlockSpec((BN, HC), lambda i: (i, 0)),
            pl.BlockSpec((BN, 1), lambda i: (i, 0)),
            const((HC, HC)), const((1, HC)),
            const((HC, HC)), const((1, HC)),
            const((HC, HC)), const((1, HC)),
            const((HC, HC)), const((1, HC)),
            const((HC, 1)), const((1, 1)),
        ],
        out_specs=pl.BlockSpec((1, NG), lambda i: (0, 0)),
        out_shape=jax.ShapeDtypeStruct((1, NG), f32),
        scratch_shapes=[pltpu.VMEM((1, NG), f32), pltpu.VMEM((1, NG), f32)],
    )(h32, mask2, w1, b1, w1g, b1g, w2, b2, w2g, b2g, w3, b3)


def kernel(xh0, edge_index, t, conditions, n_frag_switch, combined_mask,
           edge_attr, params):
    p = params
    feat = xh0[:, 3:]
    pos_pad = jnp.pad(xh0[:, :3], ((0, 0), (0, PD - 3)))
    t2 = t.reshape(1, 1)
    src = edge_index[0]
    dst = edge_index[1]

    w2p = jnp.pad(p['enc_W2'], ((0, 0), (0, 1)))
    b2p = jnp.pad(p['enc_b2'], (0, 1)).reshape(1, HC)
    e127 = p['emb_W'][HC - 1:HC, :]

    h32, pos_t = _pre_call(feat, pos_pad, t2,
                           p['enc_W1'], p['enc_b1'].reshape(1, 256),
                           w2p, b2p,
                           p['emb_W'], e127, p['emb_b'].reshape(1, HC))

    zrows = jnp.zeros((RB, HC), f32)
    for l in range(2):
        ew1 = p['l%d_eW1' % l]
        hs, hd, ps, pd_ = _gather_hp(h32, pos_t, src, dst)
        m, wr = _edge_call(hs, hd, ps, pd_, edge_attr,
                           p['ee_W1'], p['ee_b1'].reshape(1, 2 * ENF),
                           p['ee_W2'], p['ee_b2'].reshape(1, ENF),
                           ew1[:HC], ew1[HC:2 * HC], ew1[2 * HC + 1:],
                           ew1[2 * HC:2 * HC + 1],
                           p['l%d_eb1' % l].reshape(1, HC),
                           p['l%d_eW2' % l], p['l%d_eb2' % l].reshape(1, HC),
                           p['l%d_xW' % l], p['l%d_xb' % l].reshape(1, 1))
        aggf = _scatter_m(m, dst, zrows)
        posdf = _scatter_m(wr, dst, zrows)
        hw1 = p['l%d_hW1' % l]
        h32, pos_t = _node_call(h32, pos_t,
                                aggf.reshape(NC, N, HC),
                                posdf.reshape(NC, N, HC),
                                hw1[:HC], hw1[HC:],
                                p['l%d_hb1' % l].reshape(1, HC),
                                p['l%d_hW2' % l],
                                p['l%d_hb2' % l].reshape(1, HC))

    conf = _ro_call(h32, combined_mask.reshape(N, 1),
                    p['ro_W1'], p['ro_b1'].reshape(1, HC),
                    p['ro_W1g'], p['ro_b1g'].reshape(1, HC),
                    p['ro_W2'], p['ro_b2'].reshape(1, HC),
                    p['ro_W2g'], p['ro_b2g'].reshape(1, HC),
                    p['ro_W3'], p['ro_b3'].reshape(1, 1))
    return conf.reshape(NG, 1)


# speedup vs baseline: 2.3781x; 1.1009x over previous
import functools
import jax
import jax.numpy as jnp
from jax import lax
from jax.experimental import pallas as pl
from jax.experimental.pallas import tpu as pltpu
from jax.experimental.pallas import tpu_sc as plsc

N = 10000
E = 320000
HC = 128
ENF = 16
NG = 16
PD = 16
NC = 2
NS = 16
NW = NC * NS
EPW = E // NW
CHG = 200
NITG = EPW // CHG
CH = 80
RB = 632
RBL = N - (NS - 1) * RB

BN = 2000
BE = 2560

f32 = jnp.float32


def _swish(x):
    return x * jax.nn.sigmoid(x)


def _dot(a, b):
    return jnp.dot(a, b, preferred_element_type=f32)


def _mesh():
    return plsc.VectorSubcoreMesh(core_axis_name="c", subcore_axis_name="s")


def _pre_body(feat_ref, pos_ref, t_ref, w1, b1, w2, b2, ew, e127, eb,
              h_ref, p_ref):
    z = _swish(_dot(feat_ref[...], w1[...]) + b1[...])
    hp = _dot(z, w2[...]) + b2[...]
    h_ref[...] = _dot(hp, ew[...]) + t_ref[0, 0] * e127[...] + eb[...]
    p_ref[...] = pos_ref[...]


def _pre_call(feat, pos_pad, t2, w1, b1, w2p, b2p, ew, e127, eb):
    g = N // BN
    const = lambda shape: pl.BlockSpec(shape, lambda i: (0, 0))
    return pl.pallas_call(
        _pre_body,
        grid=(g,),
        in_specs=[
            pl.BlockSpec((BN, HC), lambda i: (i, 0)),
            pl.BlockSpec((BN, PD), lambda i: (i, 0)),
            pl.BlockSpec(memory_space=pltpu.SMEM),
            const((HC, 256)), const((1, 256)),
            const((256, HC)), const((1, HC)),
            const((HC, HC)), const((1, HC)), const((1, HC)),
        ],
        out_specs=[pl.BlockSpec((BN, HC), lambda i: (i, 0)),
                   pl.BlockSpec((BN, PD), lambda i: (i, 0))],
        out_shape=[jax.ShapeDtypeStruct((N, HC), f32),
                   jax.ShapeDtypeStruct((N, PD), f32)],
    )(feat, pos_pad, t2, w1, b1, w2p, b2p, ew, e127, eb)


@functools.partial(
    pl.kernel,
    out_type=(jax.ShapeDtypeStruct((E, HC), f32),
              jax.ShapeDtypeStruct((E, HC), f32),
              jax.ShapeDtypeStruct((E, HC), f32),
              jax.ShapeDtypeStruct((E, HC), f32)),
    mesh=_mesh(),
    scratch_types=(pltpu.VMEM((EPW,), jnp.int32),
                   pltpu.VMEM((2, CHG, HC), f32),
                   pltpu.VMEM((2, CHG, PD), f32),
                   pltpu.SemaphoreType.DMA((2,)),
                   pltpu.SemaphoreType.DMA((2,))),
    compiler_params=pltpu.CompilerParams(use_tc_tiling_on_sc=False),
)
def _gather_hp(h_ref, p_ref, src_ref, dst_ref,
               ohs_ref, ohd_ref, ops_ref, opd_ref,
               idxb, bufh, bufp, gsem, osem):
    wid = lax.axis_index("s") * NC + lax.axis_index("c")
    base0 = wid * EPW

    def phase(idx_hbm, oh_hbm, op_hbm):
        pltpu.sync_copy(idx_hbm.at[pl.ds(base0, EPW)], idxb)

        def descs(k, p):
            isl = idxb.at[pl.ds(k * CHG, CHG)]
            rsl = pl.ds(base0 + k * CHG, CHG)
            return (
                pltpu.make_async_copy(h_ref.at[isl], bufh.at[p], gsem.at[p]),
                pltpu.make_async_copy(p_ref.at[isl], bufp.at[p], gsem.at[p]),
                pltpu.make_async_copy(bufh.at[p], oh_hbm.at[rsl], osem.at[p]),
                pltpu.make_async_copy(bufp.at[p],
                                      op_hbm.at[rsl, pl.ds(0, PD)],
                                      osem.at[p]),
            )

        def g_start(k, p):
            gh, gp, _, _ = descs(k, p)
            gh.start()
            gp.start()

        def o_wait(k):
            _, _, oh, op = descs(k, k % 2)
            oh.wait()
            op.wait()

        g_start(0, 0)

        def body(k, carry):
            p = k % 2

            @pl.when(k + 1 < NITG)
            def _():
                @pl.when(k >= 1)
                def _():
                    o_wait(k - 1)
                g_start(k + 1, 1 - p)

            gh, gp, oh, op = descs(k, p)
            gh.wait()
            gp.wait()
            oh.start()
            op.start()
            return carry

        lax.fori_loop(0, NITG, body, 0)
        o_wait(NITG - 2)
        o_wait(NITG - 1)

    phase(src_ref, ohs_ref, ops_ref)
    phase(dst_ref, ohd_ref, opd_ref)


def _tile_rows(s):
    base = s * RB
    size = jnp.where(s == NS - 1, RBL, RB)
    return base, size


@functools.partial(
    pl.kernel,
    out_type=(jax.ShapeDtypeStruct((NC * N, HC), f32),
              jax.ShapeDtypeStruct((NC * N, PD), f32)),
    mesh=_mesh(),
    scratch_types=(pltpu.VMEM((2, CH), jnp.int32),
                   pltpu.VMEM((2, CH, HC), f32),
                   pltpu.VMEM((2, CH, PD), f32),
                   pltpu.SemaphoreType.DMA((2,)),
                   pltpu.SemaphoreType.DMA((2,)),
                   pltpu.VMEM_SHARED((N, HC), f32),
                   pltpu.VMEM_SHARED((N, PD), f32)),
    compiler_params=pltpu.CompilerParams(use_tc_tiling_on_sc=False),
)
def _scatter_mw(m_ref, wr_ref, dst_ref, z_ref, aggo_ref, posdo_ref,
                ib, rbm, rbw, lsem_i, lsem_r, accm, accw):
    c = lax.axis_index("c")
    s = lax.axis_index("s")
    wid = s * NC + c
    rbase, rsize = _tile_rows(s)
    pltpu.sync_copy(z_ref.at[pl.ds(0, rsize)], accm.at[pl.ds(rbase, rsize)])
    pltpu.sync_copy(z_ref.at[pl.ds(0, rsize), pl.ds(0, PD)],
                    accw.at[pl.ds(rbase, rsize)])
    plsc.subcore_barrier()
    base0 = wid * EPW
    nit = EPW // CH

    def descs(k, p):
        b = pl.ds(base0 + k * CH, CH)
        return (
            pltpu.make_async_copy(dst_ref.at[b], ib.at[p], lsem_i.at[p]),
            pltpu.make_async_copy(m_ref.at[b], rbm.at[p], lsem_r.at[p]),
            pltpu.make_async_copy(wr_ref.at[b, pl.ds(0, PD)], rbw.at[p],
                                  lsem_r.at[p]),
        )

    def l_start(k, p):
        for d in descs(k, p):
            d.start()

    def l_wait(k, p):
        for d in descs(k, p):
            d.wait()

    l_start(0, 0)

    def body(k, carry):
        p = k % 2

        @pl.when(k + 1 < nit)
        def _():
            l_start(k + 1, 1 - p)

        l_wait(k, p)
        pltpu.sync_copy(rbm.at[p], accm.at[ib.at[p]], add=True)
        pltpu.sync_copy(rbw.at[p], accw.at[ib.at[p]], add=True)
        return carry

    lax.fori_loop(0, nit, body, 0)
    plsc.subcore_barrier()
    pltpu.sync_copy(accm.at[pl.ds(rbase, rsize)],
                    aggo_ref.at[pl.ds(c * N + rbase, rsize)])
    pltpu.sync_copy(accw.at[pl.ds(rbase, rsize)],
                    posdo_ref.at[pl.ds(c * N + rbase, rsize)])


def _edge_body(hs_ref, hd_ref, ps_ref, pd_ref, ea_ref, eew1, eeb1, eew2, eeb2,
               w1s, w1d, w1e, w1d2, b1, w2, b2, xw, xb, m_ref, wr_ref):
    rel = ps_ref[:, :PD] - pd_ref[:, :PD]
    d2 = jnp.sum(rel * rel, axis=1, keepdims=True)
    e = _dot(_swish(_dot(ea_ref[...], eew1[...]) + eeb1[...]), eew2[...]) + eeb2[...]
    pre = (_dot(hs_ref[...], w1s[...]) + _dot(hd_ref[...], w1d[...])
           + _dot(e, w1e[...]) + d2 * w1d2[...] + b1[...])
    m = _swish(_dot(_swish(pre), w2[...]) + b2[...])
    coef = _dot(m, xw[...]) + xb[...]
    m_ref[...] = m
    wr_ref[:, :PD] = rel * (coef / (jnp.sqrt(d2) + 1.0))
    wr_ref[:, PD:] = jnp.zeros((wr_ref.shape[0], HC - PD), f32)


def _edge_call(hs, hd, ps, pd, edge_attr, eew1, eeb1, eew2, eeb2,
               w1s, w1d, w1e, w1d2, b1, w2, b2, xw, xb):
    g = E // BE
    const = lambda shape: pl.BlockSpec(shape, lambda i: (0, 0))
    return pl.pallas_call(
        _edge_body,
        grid=(g,),
        in_specs=[
            pl.BlockSpec((BE, HC), lambda i: (i, 0)),
            pl.BlockSpec((BE, HC), lambda i: (i, 0)),
            pl.BlockSpec((BE, HC), lambda i: (i, 0)),
            pl.BlockSpec((BE, HC), lambda i: (i, 0)),
            pl.BlockSpec((BE, ENF), lambda i: (i, 0)),
            const((ENF, 2 * ENF)), const((1, 2 * ENF)),
            const((2 * ENF, ENF)), const((1, ENF)),
            const((HC, HC)), const((HC, HC)), const((ENF, HC)),
            const((1, HC)), const((1, HC)),
            const((HC, HC)), const((1, HC)),
            const((HC, 1)), const((1, 1)),
        ],
        out_specs=[pl.BlockSpec((BE, HC), lambda i: (i, 0)),
                   pl.BlockSpec((BE, HC), lambda i: (i, 0))],
        out_shape=[jax.ShapeDtypeStruct((E, HC), f32),
                   jax.ShapeDtypeStruct((E, HC), f32)],
    )(hs, hd, ps, pd, edge_attr, eew1, eeb1, eew2, eeb2,
      w1s, w1d, w1e, w1d2, b1, w2, b2, xw, xb)


def _node_body(h_ref, p_ref, agg_ref, pd_ref, w1h, w1a, b1, w2, b2,
               ho_ref, po_ref):
    h = h_ref[...]
    ag = agg_ref[0] + agg_ref[1]
    u = _swish(_dot(h, w1h[...]) + _dot(ag, w1a[...]) + b1[...])
    ho_ref[...] = h + _dot(u, w2[...]) + b2[...]
    po_ref[...] = p_ref[...] + pd_ref[0] + pd_ref[1]


def _node_call(h32, pos_t, aggp, posdp, w1h, w1a, b1, w2, b2):
    g = N // BN
    const = lambda shape: pl.BlockSpec(shape, lambda i: (0, 0))
    return pl.pallas_call(
        _node_body,
        grid=(g,),
        in_specs=[
            pl.BlockSpec((BN, HC), lambda i: (i, 0)),
            pl.BlockSpec((BN, PD), lambda i: (i, 0)),
            pl.BlockSpec((NC, BN, HC), lambda i: (0, i, 0)),
            pl.BlockSpec((NC, BN, PD), lambda i: (0, i, 0)),
            const((HC, HC)), const((HC, HC)), const((1, HC)),
            const((HC, HC)), const((1, HC)),
        ],
        out_specs=[pl.BlockSpec((BN, HC), lambda i: (i, 0)),
                   pl.BlockSpec((BN, PD), lambda i: (i, 0))],
        out_shape=[jax.ShapeDtypeStruct((N, HC), f32),
                   jax.ShapeDtypeStruct((N, PD), f32)],
    )(h32, pos_t, aggp, posdp, w1h, w1a, b1, w2, b2)


def _ro_body(h_ref, mk_ref, w1, b1, w1g, b1g, w2, b2, w2g, b2g, w3, b3,
             conf_ref, s_acc, c_acc):
    i = pl.program_id(0)

    @pl.when(i == 0)
    def _():
        s_acc[...] = jnp.zeros_like(s_acc)
        c_acc[...] = jnp.zeros_like(c_acc)

    h = h_ref[...]
    g1 = jax.nn.sigmoid(_dot(h, w1g[...]) + b1g[...])
    v = _swish((_dot(h, w1[...]) + b1[...]) * g1)
    g2 = jax.nn.sigmoid(_dot(v, w2g[...]) + b2g[...])
    v = _swish((_dot(v, w2[...]) + b2[...]) * g2)
    nout = _dot(v, w3[...]) + b3[...]
    oh = (mk_ref[...] == lax.broadcasted_iota(jnp.int32, (BN, NG), 1)).astype(f32)
    s_acc[...] += jnp.sum(oh * nout, axis=0, keepdims=True)
    c_acc[...] += jnp.sum(oh, axis=0, keepdims=True)
    conf_ref[...] = s_acc[...] / jnp.maximum(c_acc[...], 1.0)


def _ro_call(h32, mask2, w1, b1, w1g, b1g, w2, b2, w2g, b2g, w3, b3):
    g = N // BN
    const = lambda shape: pl.BlockSpec(shape, lambda i: (0, 0))
    return pl.pallas_call(
        _ro_body,
        grid=(g,),
        in_specs=[
            pl.BlockSpec((BN, HC), lambda i: (i, 0)),
            pl.BlockSpec((BN, 1), lambda i: (i, 0)),
            const((HC, HC)), const((1, HC)),
            const((HC, HC)), const((1, HC)),
            const((HC, HC)), const((1, HC)),
            const((HC, HC)), const((1, HC)),
            const((HC, 1)), const((1, 1)),
        ],
        out_specs=pl.BlockSpec((1, NG), lambda i: (0, 0)),
        out_shape=jax.ShapeDtypeStruct((1, NG), f32),
        scratch_shapes=[pltpu.VMEM((1, NG), f32), pltpu.VMEM((1, NG), f32)],
    )(h32, mask2, w1, b1, w1g, b1g, w2, b2, w2g, b2g, w3, b3)


def kernel(xh0, edge_index, t, conditions, n_frag_switch, combined_mask,
           edge_attr, params):
    p = params
    feat = xh0[:, 3:]
    pos_pad = jnp.pad(xh0[:, :3], ((0, 0), (0, PD - 3)))
    t2 = t.reshape(1, 1)
    src = edge_index[0]
    dst = edge_index[1]

    w2p = jnp.pad(p['enc_W2'], ((0, 0), (0, 1)))
    b2p = jnp.pad(p['enc_b2'], (0, 1)).reshape(1, HC)
    e127 = p['emb_W'][HC - 1:HC, :]

    h32, pos_t = _pre_call(feat, pos_pad, t2,
                           p['enc_W1'], p['enc_b1'].reshape(1, 256),
                           w2p, b2p,
                           p['emb_W'], e127, p['emb_b'].reshape(1, HC))

    zrows = jnp.zeros((RB, HC), f32)
    for l in range(2):
        ew1 = p['l%d_eW1' % l]
        hs, hd, ps, pd_ = _gather_hp(h32, pos_t, src, dst)
        m, wr = _edge_call(hs, hd, ps, pd_, edge_attr,
                           p['ee_W1'], p['ee_b1'].reshape(1, 2 * ENF),
                           p['ee_W2'], p['ee_b2'].reshape(1, ENF),
                           ew1[:HC], ew1[HC:2 * HC], ew1[2 * HC + 1:],
                           ew1[2 * HC:2 * HC + 1],
                           p['l%d_eb1' % l].reshape(1, HC),
                           p['l%d_eW2' % l], p['l%d_eb2' % l].reshape(1, HC),
                           p['l%d_xW' % l], p['l%d_xb' % l].reshape(1, 1))
        aggf, posdf = _scatter_mw(m, wr, dst, zrows)
        hw1 = p['l%d_hW1' % l]
        h32, pos_t = _node_call(h32, pos_t,
                                aggf.reshape(NC, N, HC),
                                posdf.reshape(NC, N, PD),
                                hw1[:HC], hw1[HC:],
                                p['l%d_hb1' % l].reshape(1, HC),
                                p['l%d_hW2' % l],
                                p['l%d_hb2' % l].reshape(1, HC))

    conf = _ro_call(h32, combined_mask.reshape(N, 1),
                    p['ro_W1'], p['ro_b1'].reshape(1, HC),
                    p['ro_W1g'], p['ro_b1g'].reshape(1, HC),
                    p['ro_W2'], p['ro_b2'].reshape(1, HC),
                    p['ro_W2g'], p['ro_b2g'].reshape(1, HC),
                    p['ro_W3'], p['ro_b3'].reshape(1, 1))
    return conf.reshape(NG, 1)


# speedup vs baseline: 2.5305x; 1.0641x over previous
import functools
import jax
import jax.numpy as jnp
from jax import lax
from jax.experimental import pallas as pl
from jax.experimental.pallas import tpu as pltpu
from jax.experimental.pallas import tpu_sc as plsc

N = 10000
E = 320000
HC = 128
ENF = 16
NG = 16
PD = 16
NC = 2
NS = 16
NW = NC * NS
EPW = E // NW
CHG = 400
NITG = EPW // CHG
CH = 80
RB = 632
RBL = N - (NS - 1) * RB

BN = 2000
BE = 2560

f32 = jnp.float32


def _swish(x):
    return x * jax.nn.sigmoid(x)


def _dot(a, b):
    return jnp.dot(a, b, preferred_element_type=f32)


def _mesh():
    return plsc.VectorSubcoreMesh(core_axis_name="c", subcore_axis_name="s")


def _pre_body(feat_ref, pos_ref, t_ref, w1, b1, w2, b2, ew, e127, eb,
              h_ref, p_ref):
    z = _swish(_dot(feat_ref[...], w1[...]) + b1[...])
    hp = _dot(z, w2[...]) + b2[...]
    h_ref[...] = _dot(hp, ew[...]) + t_ref[0, 0] * e127[...] + eb[...]
    p_ref[...] = pos_ref[...]


def _pre_call(feat, pos_pad, t2, w1, b1, w2p, b2p, ew, e127, eb):
    g = N // BN
    const = lambda shape: pl.BlockSpec(shape, lambda i: (0, 0))
    return pl.pallas_call(
        _pre_body,
        grid=(g,),
        in_specs=[
            pl.BlockSpec((BN, HC), lambda i: (i, 0)),
            pl.BlockSpec((BN, PD), lambda i: (i, 0)),
            pl.BlockSpec(memory_space=pltpu.SMEM),
            const((HC, 256)), const((1, 256)),
            const((256, HC)), const((1, HC)),
            const((HC, HC)), const((1, HC)), const((1, HC)),
        ],
        out_specs=[pl.BlockSpec((BN, HC), lambda i: (i, 0)),
                   pl.BlockSpec((BN, PD), lambda i: (i, 0))],
        out_shape=[jax.ShapeDtypeStruct((N, HC), f32),
                   jax.ShapeDtypeStruct((N, PD), f32)],
    )(feat, pos_pad, t2, w1, b1, w2p, b2p, ew, e127, eb)


@functools.partial(
    pl.kernel,
    out_type=(jax.ShapeDtypeStruct((E, HC), f32),
              jax.ShapeDtypeStruct((E, HC), f32),
              jax.ShapeDtypeStruct((E, HC), f32),
              jax.ShapeDtypeStruct((E, HC), f32)),
    mesh=_mesh(),
    scratch_types=(pltpu.VMEM((EPW,), jnp.int32),
                   pltpu.VMEM((2, CHG, HC), f32),
                   pltpu.VMEM((2, CHG, PD), f32),
                   pltpu.SemaphoreType.DMA((2,)),
                   pltpu.SemaphoreType.DMA((2,))),
    compiler_params=pltpu.CompilerParams(use_tc_tiling_on_sc=False),
)
def _gather_hp(h_ref, p_ref, src_ref, dst_ref,
               ohs_ref, ohd_ref, ops_ref, opd_ref,
               idxb, bufh, bufp, gsem, osem):
    wid = lax.axis_index("s") * NC + lax.axis_index("c")
    base0 = wid * EPW

    def phase(idx_hbm, oh_hbm, op_hbm):
        pltpu.sync_copy(idx_hbm.at[pl.ds(base0, EPW)], idxb)

        def descs(k, p):
            isl = idxb.at[pl.ds(k * CHG, CHG)]
            rsl = pl.ds(base0 + k * CHG, CHG)
            return (
                pltpu.make_async_copy(h_ref.at[isl], bufh.at[p], gsem.at[p]),
                pltpu.make_async_copy(p_ref.at[isl], bufp.at[p], gsem.at[p]),
                pltpu.make_async_copy(bufh.at[p], oh_hbm.at[rsl], osem.at[p]),
                pltpu.make_async_copy(bufp.at[p],
                                      op_hbm.at[rsl, pl.ds(0, PD)],
                                      osem.at[p]),
            )

        def g_start(k, p):
            gh, gp, _, _ = descs(k, p)
            gh.start()
            gp.start()

        def o_wait(k):
            _, _, oh, op = descs(k, k % 2)
            oh.wait()
            op.wait()

        g_start(0, 0)

        def body(k, carry):
            p = k % 2

            @pl.when(k + 1 < NITG)
            def _():
                @pl.when(k >= 1)
                def _():
                    o_wait(k - 1)
                g_start(k + 1, 1 - p)

            gh, gp, oh, op = descs(k, p)
            gh.wait()
            gp.wait()
            oh.start()
            op.start()
            return carry

        lax.fori_loop(0, NITG, body, 0)
        o_wait(NITG - 2)
        o_wait(NITG - 1)

    phase(src_ref, ohs_ref, ops_ref)
    phase(dst_ref, ohd_ref, opd_ref)


def _tile_rows(s):
    base = s * RB
    size = jnp.where(s == NS - 1, RBL, RB)
    return base, size


def _make_scatter(with_w):
    n_in = 2 if with_w else 1
    out_type = [jax.ShapeDtypeStruct((NC * N, HC), f32)]
    scratch = [pltpu.VMEM((2, CH), jnp.int32),
               pltpu.VMEM((2, CH, HC), f32),
               pltpu.SemaphoreType.DMA((2,)),
               pltpu.SemaphoreType.DMA((2,)),
               pltpu.VMEM_SHARED((N, HC), f32)]
    if with_w:
        out_type.append(jax.ShapeDtypeStruct((NC * N, PD), f32))
        scratch += [pltpu.VMEM((2, CH, PD), f32),
                    pltpu.VMEM_SHARED((N, PD), f32)]

    @functools.partial(
        pl.kernel,
        out_type=tuple(out_type),
        mesh=_mesh(),
        scratch_types=tuple(scratch),
        compiler_params=pltpu.CompilerParams(use_tc_tiling_on_sc=False),
    )
    def scatter_k(*refs):
        vals = refs[:n_in]
        dst_ref, z_ref = refs[n_in], refs[n_in + 1]
        outs = refs[n_in + 2:n_in + 2 + n_in]
        ib, rbm, lsem_i, lsem_r, accm = refs[n_in + 2 + n_in:n_in + 7 + n_in]
        if with_w:
            rbw, accw = refs[-2:]
        c = lax.axis_index("c")
        s = lax.axis_index("s")
        wid = s * NC + c
        rbase, rsize = _tile_rows(s)
        pltpu.sync_copy(z_ref.at[pl.ds(0, rsize)],
                        accm.at[pl.ds(rbase, rsize)])
        if with_w:
            pltpu.sync_copy(z_ref.at[pl.ds(0, rsize), pl.ds(0, PD)],
                            accw.at[pl.ds(rbase, rsize)])
        plsc.subcore_barrier()
        base0 = wid * EPW
        nit = EPW // CH

        def descs(k, p):
            b = pl.ds(base0 + k * CH, CH)
            ds = [pltpu.make_async_copy(dst_ref.at[b], ib.at[p], lsem_i.at[p]),
                  pltpu.make_async_copy(vals[0].at[b], rbm.at[p],
                                        lsem_r.at[p])]
            if with_w:
                ds.append(pltpu.make_async_copy(
                    vals[1].at[b, pl.ds(0, PD)], rbw.at[p], lsem_r.at[p]))
            return ds

        def l_start(k, p):
            for d in descs(k, p):
                d.start()

        def l_wait(k, p):
            for d in descs(k, p):
                d.wait()

        l_start(0, 0)

        def body(k, carry):
            p = k % 2

            @pl.when(k + 1 < nit)
            def _():
                l_start(k + 1, 1 - p)

            l_wait(k, p)
            pltpu.sync_copy(rbm.at[p], accm.at[ib.at[p]], add=True)
            if with_w:
                pltpu.sync_copy(rbw.at[p], accw.at[ib.at[p]], add=True)
            return carry

        lax.fori_loop(0, nit, body, 0)
        plsc.subcore_barrier()
        pltpu.sync_copy(accm.at[pl.ds(rbase, rsize)],
                        outs[0].at[pl.ds(c * N + rbase, rsize)])
        if with_w:
            pltpu.sync_copy(accw.at[pl.ds(rbase, rsize)],
                            outs[1].at[pl.ds(c * N + rbase, rsize)])

    return scatter_k


_scatter_mw = _make_scatter(True)
_scatter_m = _make_scatter(False)


def _make_edge_body(with_wr):
    def body(hs_ref, hd_ref, ps_ref, pd_ref, ea_ref, eew1, eeb1, eew2, eeb2,
             w1s, w1d, w1e, w1d2, b1, w2, b2, xw, xb, m_ref, wr_ref=None):
        rel = ps_ref[:, :PD] - pd_ref[:, :PD]
        d2 = jnp.sum(rel * rel, axis=1, keepdims=True)
        e = (_dot(_swish(_dot(ea_ref[...], eew1[...]) + eeb1[...]), eew2[...])
             + eeb2[...])
        pre = (_dot(hs_ref[...], w1s[...]) + _dot(hd_ref[...], w1d[...])
               + _dot(e, w1e[...]) + d2 * w1d2[...] + b1[...])
        m = _swish(_dot(_swish(pre), w2[...]) + b2[...])
        m_ref[...] = m
        if with_wr:
            coef = _dot(m, xw[...]) + xb[...]
            wr_ref[:, :PD] = rel * (coef / (jnp.sqrt(d2) + 1.0))
            wr_ref[:, PD:] = jnp.zeros((wr_ref.shape[0], HC - PD), f32)
    return body


def _edge_call(with_wr, hs, hd, ps, pd, edge_attr, eew1, eeb1, eew2, eeb2,
               w1s, w1d, w1e, w1d2, b1, w2, b2, xw, xb):
    g = E // BE
    const = lambda shape: pl.BlockSpec(shape, lambda i: (0, 0))
    n_out = 2 if with_wr else 1
    return pl.pallas_call(
        _make_edge_body(with_wr),
        grid=(g,),
        in_specs=[
            pl.BlockSpec((BE, HC), lambda i: (i, 0)),
            pl.BlockSpec((BE, HC), lambda i: (i, 0)),
            pl.BlockSpec((BE, HC), lambda i: (i, 0)),
            pl.BlockSpec((BE, HC), lambda i: (i, 0)),
            pl.BlockSpec((BE, ENF), lambda i: (i, 0)),
            const((ENF, 2 * ENF)), const((1, 2 * ENF)),
            const((2 * ENF, ENF)), const((1, ENF)),
            const((HC, HC)), const((HC, HC)), const((ENF, HC)),
            const((1, HC)), const((1, HC)),
            const((HC, HC)), const((1, HC)),
            const((HC, 1)), const((1, 1)),
        ],
        out_specs=[pl.BlockSpec((BE, HC), lambda i: (i, 0))] * n_out,
        out_shape=[jax.ShapeDtypeStruct((E, HC), f32)] * n_out,
    )(hs, hd, ps, pd, edge_attr, eew1, eeb1, eew2, eeb2,
      w1s, w1d, w1e, w1d2, b1, w2, b2, xw, xb)


def _node_body(h_ref, p_ref, agg_ref, pd_ref, w1h, w1a, b1, w2, b2,
               ho_ref, po_ref):
    h = h_ref[...]
    ag = agg_ref[0] + agg_ref[1]
    u = _swish(_dot(h, w1h[...]) + _dot(ag, w1a[...]) + b1[...])
    ho_ref[...] = h + _dot(u, w2[...]) + b2[...]
    po_ref[...] = p_ref[...] + pd_ref[0] + pd_ref[1]


def _node_h_body(h_ref, agg_ref, w1h, w1a, b1, w2, b2, ho_ref):
    h = h_ref[...]
    ag = agg_ref[0] + agg_ref[1]
    u = _swish(_dot(h, w1h[...]) + _dot(ag, w1a[...]) + b1[...])
    ho_ref[...] = h + _dot(u, w2[...]) + b2[...]


def _node_h_call(h32, aggp, w1h, w1a, b1, w2, b2):
    g = N // BN
    const = lambda shape: pl.BlockSpec(shape, lambda i: (0, 0))
    return pl.pallas_call(
        _node_h_body,
        grid=(g,),
        in_specs=[
            pl.BlockSpec((BN, HC), lambda i: (i, 0)),
            pl.BlockSpec((NC, BN, HC), lambda i: (0, i, 0)),
            const((HC, HC)), const((HC, HC)), const((1, HC)),
            const((HC, HC)), const((1, HC)),
        ],
        out_specs=pl.BlockSpec((BN, HC), lambda i: (i, 0)),
        out_shape=jax.ShapeDtypeStruct((N, HC), f32),
    )(h32, aggp, w1h, w1a, b1, w2, b2)


def _node_call(h32, pos_t, aggp, posdp, w1h, w1a, b1, w2, b2):
    g = N // BN
    const = lambda shape: pl.BlockSpec(shape, lambda i: (0, 0))
    return pl.pallas_call(
        _node_body,
        grid=(g,),
        in_specs=[
            pl.BlockSpec((BN, HC), lambda i: (i, 0)),
            pl.BlockSpec((BN, PD), lambda i: (i, 0)),
            pl.BlockSpec((NC, BN, HC), lambda i: (0, i, 0)),
            pl.BlockSpec((NC, BN, PD), lambda i: (0, i, 0)),
            const((HC, HC)), const((HC, HC)), const((1, HC)),
            const((HC, HC)), const((1, HC)),
        ],
        out_specs=[pl.BlockSpec((BN, HC), lambda i: (i, 0)),
                   pl.BlockSpec((BN, PD), lambda i: (i, 0))],
        out_shape=[jax.ShapeDtypeStruct((N, HC), f32),
                   jax.ShapeDtypeStruct((N, PD), f32)],
    )(h32, pos_t, aggp, posdp, w1h, w1a, b1, w2, b2)


def _ro_body(h_ref, mk_ref, w1, b1, w1g, b1g, w2, b2, w2g, b2g, w3, b3,
             conf_ref, s_acc, c_acc):
    i = pl.program_id(0)

    @pl.when(i == 0)
    def _():
        s_acc[...] = jnp.zeros_like(s_acc)
        c_acc[...] = jnp.zeros_like(c_acc)

    h = h_ref[...]
    g1 = jax.nn.sigmoid(_dot(h, w1g[...]) + b1g[...])
    v = _swish((_dot(h, w1[...]) + b1[...]) * g1)
    g2 = jax.nn.sigmoid(_dot(v, w2g[...]) + b2g[...])
    v = _swish((_dot(v, w2[...]) + b2[...]) * g2)
    nout = _dot(v, w3[...]) + b3[...]
    oh = (mk_ref[...] == lax.broadcasted_iota(jnp.int32, (BN, NG), 1)).astype(f32)
    s_acc[...] += jnp.sum(oh * nout, axis=0, keepdims=True)
    c_acc[...] += jnp.sum(oh, axis=0, keepdims=True)
    conf_ref[...] = s_acc[...] / jnp.maximum(c_acc[...], 1.0)


def _ro_call(h32, mask2, w1, b1, w1g, b1g, w2, b2, w2g, b2g, w3, b3):
    g = N // BN
    const = lambda shape: pl.BlockSpec(shape, lambda i: (0, 0))
    return pl.pallas_call(
        _ro_body,
        grid=(g,),
        in_specs=[
            pl.BlockSpec((BN, HC), lambda i: (i, 0)),
            pl.BlockSpec((BN, 1), lambda i: (i, 0)),
            const((HC, HC)), const((1, HC)),
            const((HC, HC)), const((1, HC)),
            const((HC, HC)), const((1, HC)),
            const((HC, HC)), const((1, HC)),
            const((HC, 1)), const((1, 1)),
        ],
        out_specs=pl.BlockSpec((1, NG), lambda i: (0, 0)),
        out_shape=jax.ShapeDtypeStruct((1, NG), f32),
        scratch_shapes=[pltpu.VMEM((1, NG), f32), pltpu.VMEM((1, NG), f32)],
    )(h32, mask2, w1, b1, w1g, b1g, w2, b2, w2g, b2g, w3, b3)


def kernel(xh0, edge_index, t, conditions, n_frag_switch, combined_mask,
           edge_attr, params):
    p = params
    feat = xh0[:, 3:]
    pos_pad = jnp.pad(xh0[:, :3], ((0, 0), (0, PD - 3)))
    t2 = t.reshape(1, 1)
    src = edge_index[0]
    dst = edge_index[1]

    w2p = jnp.pad(p['enc_W2'], ((0, 0), (0, 1)))
    b2p = jnp.pad(p['enc_b2'], (0, 1)).reshape(1, HC)
    e127 = p['emb_W'][HC - 1:HC, :]

    h32, pos_t = _pre_call(feat, pos_pad, t2,
                           p['enc_W1'], p['enc_b1'].reshape(1, 256),
                           w2p, b2p,
                           p['emb_W'], e127, p['emb_b'].reshape(1, HC))

    zrows = jnp.zeros((RB, HC), f32)
    for l in range(2):
        last = l == 1
        ew1 = p['l%d_eW1' % l]
        hs, hd, ps, pd_ = _gather_hp(h32, pos_t, src, dst)
        eouts = _edge_call(not last, hs, hd, ps, pd_, edge_attr,
                           p['ee_W1'], p['ee_b1'].reshape(1, 2 * ENF),
                           p['ee_W2'], p['ee_b2'].reshape(1, ENF),
                           ew1[:HC], ew1[HC:2 * HC], ew1[2 * HC + 1:],
                           ew1[2 * HC:2 * HC + 1],
                           p['l%d_eb1' % l].reshape(1, HC),
                           p['l%d_eW2' % l], p['l%d_eb2' % l].reshape(1, HC),
                           p['l%d_xW' % l], p['l%d_xb' % l].reshape(1, 1))
        hw1 = p['l%d_hW1' % l]
        hb1 = p['l%d_hb1' % l].reshape(1, HC)
        hw2 = p['l%d_hW2' % l]
        hb2 = p['l%d_hb2' % l].reshape(1, HC)
        if last:
            aggf = _scatter_m(eouts[0], dst, zrows)
            if isinstance(aggf, (tuple, list)):
                aggf = aggf[0]
            h32 = _node_h_call(h32, aggf.reshape(NC, N, HC),
                               hw1[:HC], hw1[HC:], hb1, hw2, hb2)
        else:
            m, wr = eouts
            aggf, posdf = _scatter_mw(m, wr, dst, zrows)
            h32, pos_t = _node_call(h32, pos_t,
                                    aggf.reshape(NC, N, HC),
                                    posdf.reshape(NC, N, PD),
                                    hw1[:HC], hw1[HC:], hb1, hw2, hb2)

    conf = _ro_call(h32, combined_mask.reshape(N, 1),
                    p['ro_W1'], p['ro_b1'].reshape(1, HC),
                    p['ro_W1g'], p['ro_b1g'].reshape(1, HC),
                    p['ro_W2'], p['ro_b2'].reshape(1, HC),
                    p['ro_W2g'], p['ro_b2g'].reshape(1, HC),
                    p['ro_W3'], p['ro_b3'].reshape(1, 1))
    return conf.reshape(NG, 1)
